# Initial kernel scaffold; baseline (speedup 1.0000x reference)
import jax
import jax.numpy as jnp
from jax.experimental import pallas as pl


def kernel(hidden_states, seq_norm_w, ffn_norm_w, seq_router_w, ffn_router_w, Wq, Wk, Wv, Wo, W1, b1, W2, b2):
    raise NotImplementedError("write your pallas kernel here")


# speedup vs baseline: 2.1534x; 2.1534x over previous
import functools

import jax
import jax.numpy as jnp
import numpy as np
from jax import lax
from jax.experimental import pallas as pl
from jax.experimental.pallas import tpu as pltpu
from jax.experimental.pallas import tpu_sc as plsc

B, S, D = 4, 8192, 768
H, KVH, HD = 12, 4, 64
DFF = 3072
K = 1024
EPS = 1e-6
THETA = 10000.0

NC, NS, L = 2, 16, 16
NW = NC * NS
RPT = (B * K) // NW

_MESH = dict(core_axis_name="c", subcore_axis_name="s", num_cores=NC,
             num_subcores=NS)


def _prsqrt(r):
    y = lax.rsqrt(r)
    y = y * (1.5 - 0.5 * r * y * y)
    return y * (1.5 - 0.5 * r * y * y)

def _bf(z):
    return z.astype(jnp.bfloat16).astype(jnp.float32)


def _scores_body(x_ref, v_ref, s1_ref, s2_ref):
    x = x_ref[...]
    wn1 = v_ref[0]
    r1 = _bf(v_ref[1])
    wn2 = v_ref[2]
    r2 = _bf(v_ref[3])
    ssq = jnp.sum(x * x, axis=-1)
    rs1 = _prsqrt(ssq * (1.0 / D) + EPS)
    h1 = _bf(x * rs1[..., None] * wn1[None, None, :])
    s1_ref[...] = jnp.sum(h1 * r1[None, None, :], axis=-1)
    rs2 = _prsqrt(ssq * (4.0 / D) + EPS)
    h2 = _bf((2.0 * x) * rs2[..., None] * wn2[None, None, :])
    s2_ref[...] = jnp.sum(h2 * r2[None, None, :], axis=-1)


def _scores(x, vmat):
    BS = 512
    return pl.pallas_call(
        _scores_body,
        grid=(S // BS,),
        in_specs=[
            pl.BlockSpec((B, BS, D), lambda i: (0, i, 0)),
            pl.BlockSpec((8, D), lambda i: (0, 0)),
        ],
        out_specs=[
            pl.BlockSpec((B, BS), lambda i: (0, i)),
            pl.BlockSpec((B, BS), lambda i: (0, i)),
        ],
        out_shape=[
            jax.ShapeDtypeStruct((B, S), jnp.float32),
            jax.ShapeDtypeStruct((B, S), jnp.float32),
        ],
    )(x, vmat)


def _keys_from_scores(sbuf, kbuf):
    def cvt(i, _):
        xv = sbuf[pl.ds(i * L, L)]
        u = plsc.bitcast(xv, jnp.uint32)
        neg = (u >> jnp.uint32(31)) == jnp.uint32(1)
        kbuf[pl.ds(i * L, L)] = jnp.where(neg, ~u, u | jnp.uint32(0x80000000))
        return 0
    lax.fori_loop(0, S // L, cvt, 0)


def _count_ge(kbuf, t):
    def body(i, acc):
        kv = kbuf[pl.ds(i * L, L)]
        return acc + jnp.sum(jnp.where(kv >= t, 1, 0))
    return lax.fori_loop(0, S // L, body, jnp.int32(0))


def _threshold(kbuf):
    def bs(_, lohi):
        lo, hi = lohi
        mid = lo + ((hi - lo + jnp.uint32(1)) >> jnp.uint32(1))
        take = _count_ge(kbuf, mid) >= K
        return (jnp.where(take, mid, lo),
                jnp.where(take, hi, mid - jnp.uint32(1)))
    lo, _ = lax.fori_loop(0, 32, bs,
                          (jnp.uint32(0), jnp.uint32(0xFFFFFFFE)))
    cnt_gt = _count_ge(kbuf, lo + jnp.uint32(1))
    return lo, jnp.int32(K) - cnt_gt


def _compact(kbuf, ibuf, v, need_eq):
    def body(i, carry):
        off, eqc = carry
        kv = kbuf[pl.ds(i * L, L)]
        m_gt = kv > v
        m_eq = kv == v
        pref = plsc.cumsum(jnp.where(m_eq, 1, 0))
        m_take = m_eq & ((pref + eqc) <= need_eq)
        m = m_gt | m_take
        idxv = lax.broadcasted_iota(jnp.int32, (L,), 0) + i * L
        plsc.store_compressed(ibuf.at[pl.ds(off, L)], idxv, mask=m)
        npop = plsc.all_reduce_population_count(m)
        neq = plsc.all_reduce_population_count(m_take)
        return off + npop[0], eqc + neq[0]
    lax.fori_loop(0, S // L, body, (jnp.int32(0), jnp.int32(0)))


def _topk1_body(scores_hbm, idx_out, map_out, sbuf, kbuf, ibuf, mbuf, sem):
    del sem
    wid = lax.axis_index("c") * NS + lax.axis_index("s")

    @pl.when(wid < B)
    def _():
        b = wid
        pltpu.sync_copy(scores_hbm.at[b], sbuf)
        _keys_from_scores(sbuf, kbuf)
        v, need_eq = _threshold(kbuf)
        _compact(kbuf, ibuf, v, need_eq)

        def ms(i, _):
            mbuf[pl.ds(i * L, L)] = jnp.full((L,), -1, jnp.int32)
            return 0
        lax.fori_loop(0, S // L, ms, 0)

        def sc(i, _):
            iv = ibuf[pl.ds(i * L, L)]
            pv = lax.broadcasted_iota(jnp.int32, (L,), 0) + i * L
            plsc.store_scatter(mbuf, [iv], pv)
            return 0
        lax.fori_loop(0, K // L, sc, 0)

        pltpu.sync_copy(ibuf.at[pl.ds(0, K)], idx_out.at[b])
        pltpu.sync_copy(mbuf, map_out.at[pl.ds(b * S, S)])


def _topk1(scores):
    return pl.kernel(
        _topk1_body,
        out_type=[
            jax.ShapeDtypeStruct((B, K), jnp.int32),
            jax.ShapeDtypeStruct((B * S,), jnp.int32),
        ],
        mesh=plsc.VectorSubcoreMesh(**_MESH),
        compiler_params=pltpu.CompilerParams(needs_layout_passes=False),
        scratch_types=[
            pltpu.VMEM((S,), jnp.float32),
            pltpu.VMEM((S,), jnp.uint32),
            pltpu.VMEM((K + L,), jnp.int32),
            pltpu.VMEM((S,), jnp.int32),
            pltpu.SemaphoreType.DMA,
        ],
    )(scores)


def _topk2_body(scores_hbm, idx1_hbm, s2sel_hbm, idx_out,
                sbuf, kbuf, ibuf, vbuf, sem):
    del sem
    wid = lax.axis_index("c") * NS + lax.axis_index("s")

    @pl.when(wid < B)
    def _():
        b = wid
        pltpu.sync_copy(scores_hbm.at[b], sbuf)
        pltpu.sync_copy(idx1_hbm.at[b], ibuf.at[pl.ds(0, K)])
        pltpu.sync_copy(s2sel_hbm.at[b], vbuf)

        def upd(i, _):
            iv = ibuf[pl.ds(i * L, L)]
            vv = vbuf[pl.ds(i * L, L)]
            plsc.store_scatter(sbuf, [iv], vv)
            return 0
        lax.fori_loop(0, K // L, upd, 0)

        _keys_from_scores(sbuf, kbuf)
        v, need_eq = _threshold(kbuf)
        _compact(kbuf, ibuf, v, need_eq)
        pltpu.sync_copy(ibuf.at[pl.ds(0, K)], idx_out.at[b])


def _topk2(scores2b, idx1, s2sel):
    return pl.kernel(
        _topk2_body,
        out_type=jax.ShapeDtypeStruct((B, K), jnp.int32),
        mesh=plsc.VectorSubcoreMesh(**_MESH),
        compiler_params=pltpu.CompilerParams(needs_layout_passes=False),
        scratch_types=[
            pltpu.VMEM((S,), jnp.float32),
            pltpu.VMEM((S,), jnp.uint32),
            pltpu.VMEM((K + L,), jnp.int32),
            pltpu.VMEM((K,), jnp.float32),
            pltpu.SemaphoreType.DMA,
        ],
    )(scores2b, idx1, s2sel)


def _gather_body(xf_hbm, idxf_hbm, xg_out, ivb, rows, sem):
    wid = lax.axis_index("c") * NS + lax.axis_index("s")
    base = wid * RPT
    b = wid // (NW // B)
    pltpu.sync_copy(idxf_hbm.at[pl.ds(base, RPT)], ivb)

    def adj(i, _):
        ivb[pl.ds(i * L, L)] = ivb[pl.ds(i * L, L)] + b * S
        return 0
    lax.fori_loop(0, RPT // L, adj, 0)
    pltpu.async_copy(xf_hbm.at[ivb], rows, sem).wait()
    pltpu.sync_copy(rows, xg_out.at[pl.ds(base, RPT)])


def _gather_rows(xf, idxf):
    return pl.kernel(
        _gather_body,
        out_type=jax.ShapeDtypeStruct((B * K, D), jnp.float32),
        mesh=plsc.VectorSubcoreMesh(**_MESH),
        compiler_params=pltpu.CompilerParams(needs_layout_passes=False),
        scratch_types=[
            pltpu.VMEM((RPT,), jnp.int32),
            pltpu.VMEM((RPT, D), jnp.float32),
            pltpu.SemaphoreType.DMA,
        ],
    )(xf, idxf)


def _attn_body(xg_ref, wq_ref, wk_ref, wv_ref, wo_ref, cs_ref, misc_ref,
               h2_ref, s2_ref):
    x = xg_ref[0]
    ssq = jnp.sum(x * x, axis=-1, keepdims=True)
    sel = (x * _prsqrt(ssq * (1.0 / D) + EPS)
           * misc_ref[0][None, :]).astype(jnp.bfloat16)
    q = jnp.dot(sel, wq_ref[...].astype(jnp.bfloat16),
                preferred_element_type=jnp.float32)
    kk = jnp.dot(sel, wk_ref[...].astype(jnp.bfloat16),
                 preferred_element_type=jnp.float32)
    vv = jnp.dot(sel, wv_ref[...].astype(jnp.bfloat16),
                 preferred_element_type=jnp.float32)
    cos = cs_ref[:, :HD // 2]
    sin = cs_ref[:, HD // 2:]

    def rope(m):
        m1 = m[:, :HD // 2]
        m2 = m[:, HD // 2:]
        return jnp.concatenate([m1 * cos - m2 * sin,
                                m2 * cos + m1 * sin], axis=-1)

    kr = [rope(kk[:, g * HD:(g + 1) * HD]) for g in range(KVH)]
    vs = [vv[:, g * HD:(g + 1) * HD] for g in range(KVH)]
    rows = lax.broadcasted_iota(jnp.int32, (K, K), 0)
    cols = lax.broadcasted_iota(jnp.int32, (K, K), 1)
    causal = rows >= cols
    outs = []
    for h in range(H):
        qh = rope(q[:, h * HD:(h + 1) * HD]).astype(jnp.bfloat16)
        g = h // (H // KVH)
        s = lax.dot_general(qh, kr[g].astype(jnp.bfloat16),
                            (((1,), (1,)), ((), ())),
                            preferred_element_type=jnp.float32)
        s = s * (1.0 / np.sqrt(HD))
        s = jnp.where(causal, s, -1e9)
        m = jnp.max(s, axis=-1, keepdims=True)
        e = jnp.exp(s - m)
        p = (e / jnp.sum(e, axis=-1, keepdims=True)).astype(jnp.bfloat16)
        outs.append(jnp.dot(p, vs[g].astype(jnp.bfloat16),
                            preferred_element_type=jnp.float32))
    o = jnp.concatenate(outs, axis=-1).astype(jnp.bfloat16)
    attn = jnp.dot(o, wo_ref[...].astype(jnp.bfloat16),
                   preferred_element_type=jnp.float32)
    h2 = x + attn
    h2_ref[0] = h2
    ssq2 = jnp.sum(h2 * h2, axis=-1, keepdims=True)
    hn = _bf(h2 * _prsqrt(ssq2 * (1.0 / D) + EPS) * misc_ref[1][None, :])
    s2_ref[0, 0] = jnp.sum(hn * _bf(misc_ref[2])[None, :], axis=-1)


def _attention(xg, Wq, Wk, Wv, Wo, cs, misc):
    return pl.pallas_call(
        _attn_body,
        grid=(B,),
        in_specs=[
            pl.BlockSpec((1, K, D), lambda b: (b, 0, 0)),
            pl.BlockSpec((D, H * HD), lambda b: (0, 0)),
            pl.BlockSpec((D, KVH * HD), lambda b: (0, 0)),
            pl.BlockSpec((D, KVH * HD), lambda b: (0, 0)),
            pl.BlockSpec((H * HD, D), lambda b: (0, 0)),
            pl.BlockSpec((K, HD), lambda b: (0, 0)),
            pl.BlockSpec((8, D), lambda b: (0, 0)),
        ],
        out_specs=[
            pl.BlockSpec((1, K, D), lambda b: (b, 0, 0)),
            pl.BlockSpec((1, 1, K), lambda b: (b, 0, 0)),
        ],
        out_shape=[
            jax.ShapeDtypeStruct((B, K, D), jnp.float32),
            jax.ShapeDtypeStruct((B, 1, K), jnp.float32),
        ],
    )(xg, Wq, Wk, Wv, Wo, cs, misc)


def _rows2_body(xf_hbm, h2f_hbm, map_hbm, idx2f_hbm, out_hbm,
                iv2, pv, ivx, ivh, xbuf, hbuf, sem):
    wid = lax.axis_index("c") * NS + lax.axis_index("s")
    base = wid * RPT
    b = wid // (NW // B)
    CH = 64
    pltpu.sync_copy(idx2f_hbm.at[pl.ds(base, RPT)], iv2)

    def adj(i, _):
        iv2[pl.ds(i * L, L)] = iv2[pl.ds(i * L, L)] + b * S
        return 0
    lax.fori_loop(0, RPT // L, adj, 0)
    pltpu.async_copy(map_hbm.at[iv2], pv, sem).wait()

    for chunk in range(RPT // CH):
        for j in range(CH // L):
            tv = iv2[pl.ds(chunk * CH + j * L, L)]
            mv = pv[pl.ds(chunk * CH + j * L, L)]
            ivx[pl.ds(j * L, L)] = tv
            ivh[pl.ds(j * L, L)] = jnp.maximum(mv, 0) + b * K
        pltpu.async_copy(xf_hbm.at[ivx], xbuf, sem).wait()
        pltpu.async_copy(h2f_hbm.at[ivh], hbuf, sem).wait()

        def mix(g, _):
            mvec = pv[pl.ds(chunk * CH + g * L, L)]
            for jj in range(L):
                j = g * L + jj
                msk = jnp.broadcast_to(mvec[jj] >= 0, (L,))

                def dloop(d, _, j=j, msk=msk):
                    hv = hbuf[j, pl.ds(d * L, L)]
                    xv = xbuf[j, pl.ds(d * L, L)]
                    xbuf[j, pl.ds(d * L, L)] = jnp.where(msk, hv, 2.0 * xv)
                    return 0
                lax.fori_loop(0, D // L, dloop, 0)
            return 0
        lax.fori_loop(0, CH // L, mix, 0)
        pltpu.sync_copy(xbuf, out_hbm.at[pl.ds(base + chunk * CH, CH)])


def _gather_rows2(xf, h2f, map1, idx2f):
    CH = 64
    return pl.kernel(
        _rows2_body,
        out_type=jax.ShapeDtypeStruct((B * K, D), jnp.float32),
        mesh=plsc.VectorSubcoreMesh(**_MESH),
        compiler_params=pltpu.CompilerParams(needs_layout_passes=False),
        scratch_types=[
            pltpu.VMEM((RPT,), jnp.int32),
            pltpu.VMEM((RPT,), jnp.int32),
            pltpu.VMEM((CH,), jnp.int32),
            pltpu.VMEM((CH,), jnp.int32),
            pltpu.VMEM((CH, D), jnp.float32),
            pltpu.VMEM((CH, D), jnp.float32),
            pltpu.SemaphoreType.DMA,
        ],
    )(xf, h2f, map1, idx2f)


def _ffn_body(r_ref, w1_ref, b1_ref, w2_ref, misc_ref, out_ref):
    r = r_ref[0]
    ssq = jnp.sum(r * r, axis=-1, keepdims=True)
    hn = (r * _prsqrt(ssq * (1.0 / D) + EPS)
          * misc_ref[0][None, :]).astype(jnp.bfloat16)
    a = jnp.dot(hn, w1_ref[...].astype(jnp.bfloat16),
                preferred_element_type=jnp.float32)
    a = a + b1_ref[0][None, :]
    sl = (a * (1.0 / (1.0 + jnp.exp(-a)))).astype(jnp.bfloat16)
    f = jnp.dot(sl, w2_ref[...].astype(jnp.bfloat16),
                preferred_element_type=jnp.float32)
    out_ref[0] = r + f + misc_ref[1][None, :]


def _ffn(rows2, W1, b1, W2, misc2):
    KC = 512
    return pl.pallas_call(
        _ffn_body,
        grid=(B, K // KC),
        in_specs=[
            pl.BlockSpec((1, KC, D), lambda b, j: (b, j, 0)),
            pl.BlockSpec((D, DFF), lambda b, j: (0, 0)),
            pl.BlockSpec((1, DFF), lambda b, j: (0, 0)),
            pl.BlockSpec((DFF, D), lambda b, j: (0, 0)),
            pl.BlockSpec((8, D), lambda b, j: (0, 0)),
        ],
        out_specs=pl.BlockSpec((1, KC, D), lambda b, j: (b, j, 0)),
        out_shape=jax.ShapeDtypeStruct((B, K, D), jnp.float32),
    )(rows2, W1, b1, W2, misc2)


def _final_body(xf_hbm, h2f_hbm, or2_hbm, idx1f_hbm, idx2f_hbm, out_hbm,
                abuf, ivb, ivc, sem):
    del sem
    c = lax.axis_index("c")
    s = lax.axis_index("s")
    CH = 32
    ROWS = 2 * S // NS
    g0 = (2 * c) * S + s * ROWS

    def phase_a(q, _):
        r0 = g0 + q * CH
        pltpu.sync_copy(xf_hbm.at[pl.ds(r0, CH)], abuf)

        def rows_loop(j, _):
            def dloop(d, _):
                abuf[j, pl.ds(d * L, L)] = abuf[j, pl.ds(d * L, L)] * 4.0
                return 0
            lax.fori_loop(0, D // L, dloop, 0)
            return 0
        lax.fori_loop(0, CH, rows_loop, 0)
        pltpu.sync_copy(abuf, out_hbm.at[pl.ds(r0, CH)])
        return 0
    lax.fori_loop(0, ROWS // CH, phase_a, 0)

    plsc.subcore_barrier()

    e0 = (2 * c) * K + s * (2 * K // NS)
    b = 2 * c + s // (NS // 2)
    pltpu.sync_copy(idx1f_hbm.at[pl.ds(e0, 2 * K // NS)], ivb)

    def adj1(i, _):
        ivb[pl.ds(i * L, L)] = ivb[pl.ds(i * L, L)] + b * S
        return 0
    lax.fori_loop(0, (2 * K // NS) // L, adj1, 0)

    for t in range((2 * K // NS) // CH):
        pltpu.sync_copy(h2f_hbm.at[pl.ds(e0 + t * CH, CH)], abuf)

        def scale2(j, _):
            def dloop(d, _):
                abuf[j, pl.ds(d * L, L)] = abuf[j, pl.ds(d * L, L)] * 2.0
                return 0
            lax.fori_loop(0, D // L, dloop, 0)
            return 0
        lax.fori_loop(0, CH, scale2, 0)
        for j in range(CH // L):
            ivc[pl.ds(j * L, L)] = ivb[pl.ds(t * CH + j * L, L)]
        pltpu.sync_copy(abuf, out_hbm.at[ivc])

    plsc.subcore_barrier()

    pltpu.sync_copy(idx2f_hbm.at[pl.ds(e0, 2 * K // NS)], ivb)
    lax.fori_loop(0, (2 * K // NS) // L, adj1, 0)
    for t in range((2 * K // NS) // CH):
        pltpu.sync_copy(or2_hbm.at[pl.ds(e0 + t * CH, CH)], abuf)
        for j in range(CH // L):
            ivc[pl.ds(j * L, L)] = ivb[pl.ds(t * CH + j * L, L)]
        pltpu.sync_copy(abuf, out_hbm.at[ivc])


def _final(xf, h2f, or2f, idx1f, idx2f):
    CH = 32
    return pl.kernel(
        _final_body,
        out_type=jax.ShapeDtypeStruct((B * S, D), jnp.float32),
        mesh=plsc.VectorSubcoreMesh(**_MESH),
        compiler_params=pltpu.CompilerParams(needs_layout_passes=False),
        scratch_types=[
            pltpu.VMEM((CH, D), jnp.float32),
            pltpu.VMEM((2 * K // NS,), jnp.int32),
            pltpu.VMEM((CH,), jnp.int32),
            pltpu.SemaphoreType.DMA,
        ],
    )(xf, h2f, or2f, idx1f, idx2f)


def kernel(hidden_states, seq_norm_w, ffn_norm_w, seq_router_w, ffn_router_w,
           Wq, Wk, Wv, Wo, W1, b1, W2, b2):
    x = hidden_states
    xf = x.reshape(B * S, D)

    vmat = (jnp.zeros((8, D), jnp.float32).at[0].set(seq_norm_w)
            .at[1].set(seq_router_w[:, 0]).at[2].set(ffn_norm_w)
            .at[3].set(ffn_router_w[:, 0]))
    misc = (jnp.zeros((8, D), jnp.float32).at[0].set(seq_norm_w)
            .at[1].set(ffn_norm_w).at[2].set(ffn_router_w[:, 0]))
    misc2 = jnp.zeros((8, D), jnp.float32).at[0].set(ffn_norm_w).at[1].set(b2)

    half = HD // 2
    inv = 1.0 / (THETA ** (jnp.arange(0, half, dtype=jnp.float32) / half))
    ang = jnp.arange(K, dtype=jnp.float32)[:, None] * inv[None, :]
    cs = jnp.concatenate([jnp.cos(ang), jnp.sin(ang)], axis=-1)

    scores1, scores2b = _scores(x, vmat)
    idx1, map1 = _topk1(scores1)
    xg = _gather_rows(xf, idx1.reshape(B * K))
    h2sel, s2sel = _attention(xg.reshape(B, K, D), Wq, Wk, Wv, Wo, cs, misc)
    idx2 = _topk2(scores2b, idx1, s2sel.reshape(B, K))
    h2f = h2sel.reshape(B * K, D)
    rows2 = _gather_rows2(xf, h2f, map1, idx2.reshape(B * K))
    outrows2 = _ffn(rows2.reshape(B, K, D), W1, b1.reshape(1, DFF), W2, misc2)
    out = _final(xf, h2f, outrows2.reshape(B * K, D),
                 idx1.reshape(B * K), idx2.reshape(B * K))
    return out.reshape(B, S, D)


# speedup vs baseline: 3.2450x; 1.5069x over previous
import functools

import jax
import jax.numpy as jnp
import numpy as np
from jax import lax
from jax.experimental import pallas as pl
from jax.experimental.pallas import tpu as pltpu
from jax.experimental.pallas import tpu_sc as plsc

B, S, D = 4, 8192, 768
H, KVH, HD = 12, 4, 64
DFF = 3072
K = 1024
EPS = 1e-6
THETA = 10000.0

NC, NS, L = 2, 16, 16
NW = NC * NS
RPT = (B * K) // NW

_MESH = dict(core_axis_name="c", subcore_axis_name="s", num_cores=NC,
             num_subcores=NS)


def _prsqrt(r):
    y = lax.rsqrt(r)
    y = y * (1.5 - 0.5 * r * y * y)
    return y * (1.5 - 0.5 * r * y * y)

def _bf(z):
    return z.astype(jnp.bfloat16).astype(jnp.float32)


def _scores_body(x_ref, v_ref, s1_ref, s2_ref):
    x = x_ref[...]
    wn1 = v_ref[0]
    r1 = _bf(v_ref[1])
    wn2 = v_ref[2]
    r2 = _bf(v_ref[3])
    ssq = jnp.sum(x * x, axis=-1)
    rs1 = _prsqrt(ssq * (1.0 / D) + EPS)
    h1 = _bf(x * rs1[..., None] * wn1[None, None, :])
    s1_ref[...] = jnp.sum(h1 * r1[None, None, :], axis=-1)
    rs2 = _prsqrt(ssq * (4.0 / D) + EPS)
    h2 = _bf((2.0 * x) * rs2[..., None] * wn2[None, None, :])
    s2_ref[...] = jnp.sum(h2 * r2[None, None, :], axis=-1)


def _scores(x, vmat):
    BS = 512
    return pl.pallas_call(
        _scores_body,
        grid=(S // BS,),
        in_specs=[
            pl.BlockSpec((B, BS, D), lambda i: (0, i, 0)),
            pl.BlockSpec((8, D), lambda i: (0, 0)),
        ],
        out_specs=[
            pl.BlockSpec((B, BS), lambda i: (0, i)),
            pl.BlockSpec((B, BS), lambda i: (0, i)),
        ],
        out_shape=[
            jax.ShapeDtypeStruct((B, S), jnp.float32),
            jax.ShapeDtypeStruct((B, S), jnp.float32),
        ],
    )(x, vmat)


def _keys_from_scores(sbuf, kbuf):
    def cvt(i, _):
        xv = sbuf[pl.ds(i * L, L)]
        u = plsc.bitcast(xv, jnp.uint32)
        neg = (u >> jnp.uint32(31)) == jnp.uint32(1)
        kbuf[pl.ds(i * L, L)] = jnp.where(neg, ~u, u | jnp.uint32(0x80000000))
        return 0
    lax.fori_loop(0, S // L, cvt, 0)


def _count_ge(kbuf, t):
    def body(i, acc):
        kv = kbuf[pl.ds(i * L, L)]
        return acc + jnp.sum(jnp.where(kv >= t, 1, 0))
    return lax.fori_loop(0, S // L, body, jnp.int32(0))


def _threshold(kbuf):
    def bs(_, lohi):
        lo, hi = lohi
        mid = lo + ((hi - lo + jnp.uint32(1)) >> jnp.uint32(1))
        take = _count_ge(kbuf, mid) >= K
        return (jnp.where(take, mid, lo),
                jnp.where(take, hi, mid - jnp.uint32(1)))
    lo, _ = lax.fori_loop(0, 32, bs,
                          (jnp.uint32(0), jnp.uint32(0xFFFFFFFE)))
    cnt_gt = _count_ge(kbuf, lo + jnp.uint32(1))
    return lo, jnp.int32(K) - cnt_gt


def _compact(kbuf, ibuf, v, need_eq):
    def body(i, carry):
        off, eqc = carry
        kv = kbuf[pl.ds(i * L, L)]
        m_gt = kv > v
        m_eq = kv == v
        pref = plsc.cumsum(jnp.where(m_eq, 1, 0))
        m_take = m_eq & ((pref + eqc) <= need_eq)
        m = m_gt | m_take
        idxv = lax.broadcasted_iota(jnp.int32, (L,), 0) + i * L
        plsc.store_compressed(ibuf.at[pl.ds(off, L)], idxv, mask=m)
        npop = plsc.all_reduce_population_count(m)
        neq = plsc.all_reduce_population_count(m_take)
        return off + npop[0], eqc + neq[0]
    lax.fori_loop(0, S // L, body, (jnp.int32(0), jnp.int32(0)))


def _topk1_body(scores_hbm, idx_out, map_out, sbuf, kbuf, ibuf, mbuf, sem):
    del sem
    wid = lax.axis_index("c") * NS + lax.axis_index("s")

    @pl.when(wid < B)
    def _():
        b = wid
        pltpu.sync_copy(scores_hbm.at[b], sbuf)
        _keys_from_scores(sbuf, kbuf)
        v, need_eq = _threshold(kbuf)
        _compact(kbuf, ibuf, v, need_eq)

        def ms(i, _):
            mbuf[pl.ds(i * L, L)] = jnp.full((L,), -1, jnp.int32)
            return 0
        lax.fori_loop(0, S // L, ms, 0)

        def sc(i, _):
            iv = ibuf[pl.ds(i * L, L)]
            pv = lax.broadcasted_iota(jnp.int32, (L,), 0) + i * L
            plsc.store_scatter(mbuf, [iv], pv)
            return 0
        lax.fori_loop(0, K // L, sc, 0)

        pltpu.sync_copy(ibuf.at[pl.ds(0, K)], idx_out.at[b])
        pltpu.sync_copy(mbuf, map_out.at[pl.ds(b * S, S)])


def _topk1(scores):
    return pl.kernel(
        _topk1_body,
        out_type=[
            jax.ShapeDtypeStruct((B, K), jnp.int32),
            jax.ShapeDtypeStruct((B * S,), jnp.int32),
        ],
        mesh=plsc.VectorSubcoreMesh(**_MESH),
        compiler_params=pltpu.CompilerParams(needs_layout_passes=False),
        scratch_types=[
            pltpu.VMEM((S,), jnp.float32),
            pltpu.VMEM((S,), jnp.uint32),
            pltpu.VMEM((K + L,), jnp.int32),
            pltpu.VMEM((S,), jnp.int32),
            pltpu.SemaphoreType.DMA,
        ],
    )(scores)


def _topk2_body(scores_hbm, idx1_hbm, s2sel_hbm, idx_out,
                sbuf, kbuf, ibuf, vbuf, sem):
    del sem
    wid = lax.axis_index("c") * NS + lax.axis_index("s")

    @pl.when(wid < B)
    def _():
        b = wid
        pltpu.sync_copy(scores_hbm.at[b], sbuf)
        pltpu.sync_copy(idx1_hbm.at[b], ibuf.at[pl.ds(0, K)])
        pltpu.sync_copy(s2sel_hbm.at[b], vbuf)

        def upd(i, _):
            iv = ibuf[pl.ds(i * L, L)]
            vv = vbuf[pl.ds(i * L, L)]
            plsc.store_scatter(sbuf, [iv], vv)
            return 0
        lax.fori_loop(0, K // L, upd, 0)

        _keys_from_scores(sbuf, kbuf)
        v, need_eq = _threshold(kbuf)
        _compact(kbuf, ibuf, v, need_eq)
        pltpu.sync_copy(ibuf.at[pl.ds(0, K)], idx_out.at[b])


def _topk2(scores2b, idx1, s2sel):
    return pl.kernel(
        _topk2_body,
        out_type=jax.ShapeDtypeStruct((B, K), jnp.int32),
        mesh=plsc.VectorSubcoreMesh(**_MESH),
        compiler_params=pltpu.CompilerParams(needs_layout_passes=False),
        scratch_types=[
            pltpu.VMEM((S,), jnp.float32),
            pltpu.VMEM((S,), jnp.uint32),
            pltpu.VMEM((K + L,), jnp.int32),
            pltpu.VMEM((K,), jnp.float32),
            pltpu.SemaphoreType.DMA,
        ],
    )(scores2b, idx1, s2sel)


def _gather_body(xf_hbm, idxf_hbm, xg_out, ivb, rows, sem):
    wid = lax.axis_index("c") * NS + lax.axis_index("s")
    base = wid * RPT
    b = wid // (NW // B)
    pltpu.sync_copy(idxf_hbm.at[pl.ds(base, RPT)], ivb)

    def adj(i, _):
        ivb[pl.ds(i * L, L)] = ivb[pl.ds(i * L, L)] + b * S
        return 0
    lax.fori_loop(0, RPT // L, adj, 0)
    pltpu.async_copy(xf_hbm.at[ivb], rows, sem).wait()
    pltpu.sync_copy(rows, xg_out.at[pl.ds(base, RPT)])


def _gather_rows(xf, idxf):
    return pl.kernel(
        _gather_body,
        out_type=jax.ShapeDtypeStruct((B * K, D), jnp.float32),
        mesh=plsc.VectorSubcoreMesh(**_MESH),
        compiler_params=pltpu.CompilerParams(needs_layout_passes=False),
        scratch_types=[
            pltpu.VMEM((RPT,), jnp.int32),
            pltpu.VMEM((RPT, D), jnp.float32),
            pltpu.SemaphoreType.DMA,
        ],
    )(xf, idxf)


def _attn_body(xg_ref, wq_ref, wk_ref, wv_ref, wo_ref, cs_ref, misc_ref,
               h2_ref, s2_ref):
    x = xg_ref[0]
    ssq = jnp.sum(x * x, axis=-1, keepdims=True)
    sel = (x * _prsqrt(ssq * (1.0 / D) + EPS)
           * misc_ref[0][None, :]).astype(jnp.bfloat16)
    q = jnp.dot(sel, wq_ref[...].astype(jnp.bfloat16),
                preferred_element_type=jnp.float32)
    kk = jnp.dot(sel, wk_ref[...].astype(jnp.bfloat16),
                 preferred_element_type=jnp.float32)
    vv = jnp.dot(sel, wv_ref[...].astype(jnp.bfloat16),
                 preferred_element_type=jnp.float32)
    cos = cs_ref[:, :HD // 2]
    sin = cs_ref[:, HD // 2:]

    def rope(m):
        m1 = m[:, :HD // 2]
        m2 = m[:, HD // 2:]
        return jnp.concatenate([m1 * cos - m2 * sin,
                                m2 * cos + m1 * sin], axis=-1)

    kr = [rope(kk[:, g * HD:(g + 1) * HD]) for g in range(KVH)]
    vs = [vv[:, g * HD:(g + 1) * HD] for g in range(KVH)]
    rows = lax.broadcasted_iota(jnp.int32, (K, K), 0)
    cols = lax.broadcasted_iota(jnp.int32, (K, K), 1)
    causal = rows >= cols
    outs = []
    for h in range(H):
        qh = rope(q[:, h * HD:(h + 1) * HD]).astype(jnp.bfloat16)
        g = h // (H // KVH)
        s = lax.dot_general(qh, kr[g].astype(jnp.bfloat16),
                            (((1,), (1,)), ((), ())),
                            preferred_element_type=jnp.float32)
        s = s * (1.0 / np.sqrt(HD))
        s = jnp.where(causal, s, -1e9)
        m = jnp.max(s, axis=-1, keepdims=True)
        e = jnp.exp(s - m)
        p = (e / jnp.sum(e, axis=-1, keepdims=True)).astype(jnp.bfloat16)
        outs.append(jnp.dot(p, vs[g].astype(jnp.bfloat16),
                            preferred_element_type=jnp.float32))
    o = jnp.concatenate(outs, axis=-1).astype(jnp.bfloat16)
    attn = jnp.dot(o, wo_ref[...].astype(jnp.bfloat16),
                   preferred_element_type=jnp.float32)
    h2 = x + attn
    h2_ref[0] = h2
    ssq2 = jnp.sum(h2 * h2, axis=-1, keepdims=True)
    hn = _bf(h2 * _prsqrt(ssq2 * (1.0 / D) + EPS) * misc_ref[1][None, :])
    s2_ref[0, 0] = jnp.sum(hn * _bf(misc_ref[2])[None, :], axis=-1)


def _attention(xg, Wq, Wk, Wv, Wo, cs, misc):
    return pl.pallas_call(
        _attn_body,
        grid=(B,),
        in_specs=[
            pl.BlockSpec((1, K, D), lambda b: (b, 0, 0)),
            pl.BlockSpec((D, H * HD), lambda b: (0, 0)),
            pl.BlockSpec((D, KVH * HD), lambda b: (0, 0)),
            pl.BlockSpec((D, KVH * HD), lambda b: (0, 0)),
            pl.BlockSpec((H * HD, D), lambda b: (0, 0)),
            pl.BlockSpec((K, HD), lambda b: (0, 0)),
            pl.BlockSpec((8, D), lambda b: (0, 0)),
        ],
        out_specs=[
            pl.BlockSpec((1, K, D), lambda b: (b, 0, 0)),
            pl.BlockSpec((1, 1, K), lambda b: (b, 0, 0)),
        ],
        out_shape=[
            jax.ShapeDtypeStruct((B, K, D), jnp.float32),
            jax.ShapeDtypeStruct((B, 1, K), jnp.float32),
        ],
    )(xg, Wq, Wk, Wv, Wo, cs, misc)


def _rows2_body(xf_hbm, h2f_hbm, map_hbm, idx2f_hbm, out_hbm,
                iv2, pv, ivx, ivh, xbuf, hbuf, sem, sem2):
    wid = lax.axis_index("c") * NS + lax.axis_index("s")
    base = wid * RPT
    b = wid // (NW // B)
    CH = 64
    pltpu.sync_copy(idx2f_hbm.at[pl.ds(base, RPT)], iv2)

    def adj(i, _):
        iv2[pl.ds(i * L, L)] = iv2[pl.ds(i * L, L)] + b * S
        return 0
    lax.fori_loop(0, RPT // L, adj, 0)
    pltpu.async_copy(map_hbm.at[iv2], pv, sem).wait()

    for chunk in range(RPT // CH):
        for j in range(CH // L):
            tv = iv2[pl.ds(chunk * CH + j * L, L)]
            mv = pv[pl.ds(chunk * CH + j * L, L)]
            ivx[pl.ds(j * L, L)] = tv
            ivh[pl.ds(j * L, L)] = jnp.maximum(mv, 0) + b * K
        cx = pltpu.async_copy(xf_hbm.at[ivx], xbuf, sem)
        ch = pltpu.async_copy(h2f_hbm.at[ivh], hbuf, sem2)
        cx.wait()
        ch.wait()

        def mix(g, _):
            mvec = pv[pl.ds(chunk * CH + g * L, L)]
            for jj in range(L):
                j = g * L + jj
                msk = jnp.broadcast_to(mvec[jj] >= 0, (L,))

                def dloop(d, _, j=j, msk=msk):
                    for u in range(4):
                        sl = pl.ds((d * 4 + u) * L, L)
                        hv = hbuf[j, sl]
                        xv = xbuf[j, sl]
                        xbuf[j, sl] = jnp.where(msk, hv, 2.0 * xv)
                    return 0
                lax.fori_loop(0, D // (4 * L), dloop, 0)
            return 0
        lax.fori_loop(0, CH // L, mix, 0)
        pltpu.sync_copy(xbuf, out_hbm.at[pl.ds(base + chunk * CH, CH)])


def _gather_rows2(xf, h2f, map1, idx2f):
    CH = 64
    return pl.kernel(
        _rows2_body,
        out_type=jax.ShapeDtypeStruct((B * K, D), jnp.float32),
        mesh=plsc.VectorSubcoreMesh(**_MESH),
        compiler_params=pltpu.CompilerParams(needs_layout_passes=False),
        scratch_types=[
            pltpu.VMEM((RPT,), jnp.int32),
            pltpu.VMEM((RPT,), jnp.int32),
            pltpu.VMEM((CH,), jnp.int32),
            pltpu.VMEM((CH,), jnp.int32),
            pltpu.VMEM((CH, D), jnp.float32),
            pltpu.VMEM((CH, D), jnp.float32),
            pltpu.SemaphoreType.DMA,
            pltpu.SemaphoreType.DMA,
        ],
    )(xf, h2f, map1, idx2f)


def _ffn_body(r_ref, w1_ref, b1_ref, w2_ref, misc_ref, out_ref):
    r = r_ref[0]
    ssq = jnp.sum(r * r, axis=-1, keepdims=True)
    hn = (r * _prsqrt(ssq * (1.0 / D) + EPS)
          * misc_ref[0][None, :]).astype(jnp.bfloat16)
    a = jnp.dot(hn, w1_ref[...].astype(jnp.bfloat16),
                preferred_element_type=jnp.float32)
    a = a + b1_ref[0][None, :]
    sl = (a * (1.0 / (1.0 + jnp.exp(-a)))).astype(jnp.bfloat16)
    f = jnp.dot(sl, w2_ref[...].astype(jnp.bfloat16),
                preferred_element_type=jnp.float32)
    out_ref[0] = r + f + misc_ref[1][None, :]


def _ffn(rows2, W1, b1, W2, misc2):
    KC = 512
    return pl.pallas_call(
        _ffn_body,
        grid=(B, K // KC),
        in_specs=[
            pl.BlockSpec((1, KC, D), lambda b, j: (b, j, 0)),
            pl.BlockSpec((D, DFF), lambda b, j: (0, 0)),
            pl.BlockSpec((1, DFF), lambda b, j: (0, 0)),
            pl.BlockSpec((DFF, D), lambda b, j: (0, 0)),
            pl.BlockSpec((8, D), lambda b, j: (0, 0)),
        ],
        out_specs=pl.BlockSpec((1, KC, D), lambda b, j: (b, j, 0)),
        out_shape=jax.ShapeDtypeStruct((B, K, D), jnp.float32),
    )(rows2, W1, b1, W2, misc2)


_FCH = 32


def _final_body(xf_hbm, h2f_hbm, or2_hbm, idx1f_hbm, idx2f_hbm, out_hbm,
                ib0, ib1, ob0, ob1, ivb, ivc0, ivc1,
                isem0, isem1, osem0, osem1):
    c = lax.axis_index("c")
    s = lax.axis_index("s")
    CH = _FCH
    ROWS = 2 * S // NS
    g0 = (2 * c) * S + s * ROWS
    NQ = ROWS // CH
    ibs, isems = (ib0, ib1), (isem0, isem1)
    obs, osems = (ob0, ob1), (osem0, osem1)

    pltpu.async_copy(xf_hbm.at[pl.ds(g0, CH)], ib0, isem0)

    def pipe(qq, _):
        for j in range(2):
            q = qq * 2 + j
            ib, isem = ibs[j], isems[j]
            ob, osem = obs[j], osems[j]
            nib, nisem = ibs[1 - j], isems[1 - j]

            @pl.when(q + 1 < NQ)
            def _():
                pltpu.async_copy(xf_hbm.at[pl.ds(g0 + (q + 1) * CH, CH)],
                                 nib, nisem)
            pltpu.make_async_copy(xf_hbm.at[pl.ds(g0, CH)], ib, isem).wait()

            @pl.when(q >= 2)
            def _():
                pltpu.make_async_copy(ob, out_hbm.at[pl.ds(g0, CH)],
                                      osem).wait()

            def rowc(r, _, ib=ib, ob=ob):
                for dd in range(D // L):
                    ob[r, pl.ds(dd * L, L)] = ib[r, pl.ds(dd * L, L)] * 4.0
                return 0
            lax.fori_loop(0, CH, rowc, 0)
            pltpu.async_copy(ob, out_hbm.at[pl.ds(g0 + q * CH, CH)], osem)
        return 0
    lax.fori_loop(0, NQ // 2, pipe, 0)
    pltpu.make_async_copy(ob0, out_hbm.at[pl.ds(g0, CH)], osem0).wait()
    pltpu.make_async_copy(ob1, out_hbm.at[pl.ds(g0, CH)], osem1).wait()

    plsc.subcore_barrier()

    NE = 2 * K // NS
    e0 = (2 * c) * K + s * NE
    b = 2 * c + s // (NS // 2)
    NT = NE // CH
    ivcs = (ivc0, ivc1)

    for phase, (src, idx_src, scale) in enumerate(
            ((h2f_hbm, idx1f_hbm, True), (or2_hbm, idx2f_hbm, False))):
        pltpu.sync_copy(idx_src.at[pl.ds(e0, NE)], ivb)

        def adj(i, _):
            ivb[pl.ds(i * L, L)] = ivb[pl.ds(i * L, L)] + b * S
            return 0
        lax.fori_loop(0, NE // L, adj, 0)

        pltpu.async_copy(src.at[pl.ds(e0, CH)], ib0, isem0)
        pltpu.async_copy(src.at[pl.ds(e0 + CH, CH)], ib1, isem1)
        for t in range(NT):
            j = t % 2
            ib, isem, osem, ivc = ibs[j], isems[j], osems[j], ivcs[j]
            pltpu.make_async_copy(src.at[pl.ds(e0, CH)], ib, isem).wait()
            if scale:
                def sc2(r, _, ib=ib):
                    for dd in range(D // L):
                        ib[r, pl.ds(dd * L, L)] = (
                            ib[r, pl.ds(dd * L, L)] * 2.0)
                    return 0
                lax.fori_loop(0, CH, sc2, 0)
            for g in range(CH // L):
                ivc[pl.ds(g * L, L)] = ivb[pl.ds(t * CH + g * L, L)]
            pltpu.async_copy(ib, out_hbm.at[ivc], osem)
            if t + 2 < NT:
                pltpu.make_async_copy(ib, out_hbm.at[ivc], osem).wait()
                pltpu.async_copy(src.at[pl.ds(e0 + (t + 2) * CH, CH)],
                                 ib, isem)
        for j in range(min(2, NT)):
            pltpu.make_async_copy(ibs[j], out_hbm.at[ivcs[j]],
                                  osems[j]).wait()
        if phase == 0:
            plsc.subcore_barrier()


def _final(xf, h2f, or2f, idx1f, idx2f):
    CH = _FCH
    return pl.kernel(
        _final_body,
        out_type=jax.ShapeDtypeStruct((B * S, D), jnp.float32),
        mesh=plsc.VectorSubcoreMesh(**_MESH),
        compiler_params=pltpu.CompilerParams(needs_layout_passes=False),
        scratch_types=[
            pltpu.VMEM((CH, D), jnp.float32),
            pltpu.VMEM((CH, D), jnp.float32),
            pltpu.VMEM((CH, D), jnp.float32),
            pltpu.VMEM((CH, D), jnp.float32),
            pltpu.VMEM((2 * K // NS,), jnp.int32),
            pltpu.VMEM((CH,), jnp.int32),
            pltpu.VMEM((CH,), jnp.int32),
            pltpu.SemaphoreType.DMA,
            pltpu.SemaphoreType.DMA,
            pltpu.SemaphoreType.DMA,
            pltpu.SemaphoreType.DMA,
        ],
    )(xf, h2f, or2f, idx1f, idx2f)


def kernel(hidden_states, seq_norm_w, ffn_norm_w, seq_router_w, ffn_router_w,
           Wq, Wk, Wv, Wo, W1, b1, W2, b2):
    x = hidden_states
    xf = x.reshape(B * S, D)

    vmat = (jnp.zeros((8, D), jnp.float32).at[0].set(seq_norm_w)
            .at[1].set(seq_router_w[:, 0]).at[2].set(ffn_norm_w)
            .at[3].set(ffn_router_w[:, 0]))
    misc = (jnp.zeros((8, D), jnp.float32).at[0].set(seq_norm_w)
            .at[1].set(ffn_norm_w).at[2].set(ffn_router_w[:, 0]))
    misc2 = jnp.zeros((8, D), jnp.float32).at[0].set(ffn_norm_w).at[1].set(b2)

    half = HD // 2
    inv = 1.0 / (THETA ** (jnp.arange(0, half, dtype=jnp.float32) / half))
    ang = jnp.arange(K, dtype=jnp.float32)[:, None] * inv[None, :]
    cs = jnp.concatenate([jnp.cos(ang), jnp.sin(ang)], axis=-1)

    scores1, scores2b = _scores(x, vmat)
    idx1, map1 = _topk1(scores1)
    xg = _gather_rows(xf, idx1.reshape(B * K))
    h2sel, s2sel = _attention(xg.reshape(B, K, D), Wq, Wk, Wv, Wo, cs, misc)
    idx2 = _topk2(scores2b, idx1, s2sel.reshape(B, K))
    h2f = h2sel.reshape(B * K, D)
    rows2 = _gather_rows2(xf, h2f, map1, idx2.reshape(B * K))
    outrows2 = _ffn(rows2.reshape(B, K, D), W1, b1.reshape(1, DFF), W2, misc2)
    out = _final(xf, h2f, outrows2.reshape(B * K, D),
                 idx1.reshape(B * K), idx2.reshape(B * K))
    return out.reshape(B, S, D)


# speedup vs baseline: 3.5715x; 1.1006x over previous
import functools

import jax
import jax.numpy as jnp
import numpy as np
from jax import lax
from jax.experimental import pallas as pl
from jax.experimental.pallas import tpu as pltpu
from jax.experimental.pallas import tpu_sc as plsc

B, S, D = 4, 8192, 768
H, KVH, HD = 12, 4, 64
DFF = 3072
K = 1024
EPS = 1e-6
THETA = 10000.0

NC, NS, L = 2, 16, 16
NW = NC * NS
RPT = (B * K) // NW

_MESH = dict(core_axis_name="c", subcore_axis_name="s", num_cores=NC,
             num_subcores=NS)


def _prsqrt(r):
    y = lax.rsqrt(r)
    y = y * (1.5 - 0.5 * r * y * y)
    return y * (1.5 - 0.5 * r * y * y)

def _bf(z):
    return z.astype(jnp.bfloat16).astype(jnp.float32)


def _scores_body(x_ref, v_ref, s1_ref, s2_ref):
    x = x_ref[...]
    wn1 = v_ref[0]
    r1 = _bf(v_ref[1])
    wn2 = v_ref[2]
    r2 = _bf(v_ref[3])
    ssq = jnp.sum(x * x, axis=-1)
    rs1 = _prsqrt(ssq * (1.0 / D) + EPS)
    h1 = _bf(x * rs1[..., None] * wn1[None, None, :])
    s1_ref[...] = jnp.sum(h1 * r1[None, None, :], axis=-1)
    rs2 = _prsqrt(ssq * (4.0 / D) + EPS)
    h2 = _bf((2.0 * x) * rs2[..., None] * wn2[None, None, :])
    s2_ref[...] = jnp.sum(h2 * r2[None, None, :], axis=-1)


def _scores(x, vmat):
    BS = 512
    return pl.pallas_call(
        _scores_body,
        grid=(S // BS,),
        in_specs=[
            pl.BlockSpec((B, BS, D), lambda i: (0, i, 0)),
            pl.BlockSpec((8, D), lambda i: (0, 0)),
        ],
        out_specs=[
            pl.BlockSpec((B, BS), lambda i: (0, i)),
            pl.BlockSpec((B, BS), lambda i: (0, i)),
        ],
        out_shape=[
            jax.ShapeDtypeStruct((B, S), jnp.float32),
            jax.ShapeDtypeStruct((B, S), jnp.float32),
        ],
    )(x, vmat)


def _keys_from_scores(sbuf, kbuf):
    def cvt(i, _):
        xv = sbuf[pl.ds(i * L, L)]
        u = plsc.bitcast(xv, jnp.uint32)
        neg = (u >> jnp.uint32(31)) == jnp.uint32(1)
        kbuf[pl.ds(i * L, L)] = jnp.where(neg, ~u, u | jnp.uint32(0x80000000))
        return 0
    lax.fori_loop(0, S // L, cvt, 0)


def _threshold(kbuf, hbuf):
    laneoff = lax.broadcasted_iota(jnp.int32, (L,), 0) * 256
    ones = jnp.full((L,), 1, jnp.int32)
    prefix = jnp.uint32(0)
    R = jnp.int32(K)
    for p in range(4):
        shift = jnp.uint32(24 - 8 * p)

        def zero(i, _):
            hbuf[pl.ds(i * L, L)] = jnp.zeros((L,), jnp.int32)
            return 0
        lax.fori_loop(0, (256 * L) // L, zero, 0)

        def build(i, _, shift=shift, prefix=prefix, p=p):
            kv = kbuf[pl.ds(i * L, L)]
            byte = ((kv >> shift) & jnp.uint32(0xFF)).astype(jnp.int32)
            idx = laneoff + byte
            if p == 0:
                plsc.addupdate_scatter(hbuf, [idx], ones)
            else:
                act = (kv >> (shift + jnp.uint32(8))) == prefix
                plsc.addupdate_scatter(hbuf, [idx], ones, mask=act)
            return 0
        lax.fori_loop(0, S // L, build, 0)

        bstar = jnp.int32(0)
        cnt_above = jnp.int32(0)
        cum_higher = jnp.int32(0)
        for g in range(15, -1, -1):
            bsum = hbuf[pl.ds(g * L, L)]
            for lane in range(1, L):
                bsum = bsum + hbuf[pl.ds(lane * 256 + g * L, L)]
            sfx = lax.rev(plsc.cumsum(lax.rev(bsum, (0,))), (0,))
            cgt = cum_higher + sfx - bsum
            cond = (cgt < R) & ((cgt + bsum) >= R)
            lanes = lax.broadcasted_iota(jnp.int32, (L,), 0) + g * L
            bstar = bstar + jnp.sum(jnp.where(cond, lanes, 0))
            cnt_above = cnt_above + jnp.sum(jnp.where(cond, cgt, 0))
            cum_higher = cum_higher + jnp.sum(bsum)
        R = R - cnt_above
        prefix = (prefix << jnp.uint32(8)) | bstar.astype(jnp.uint32)
    return prefix, R


def _compact(kbuf, ibuf, v, need_eq):
    def body(i, carry):
        off, eqc = carry
        kv = kbuf[pl.ds(i * L, L)]
        m_gt = kv > v
        m_eq = kv == v
        pref = plsc.cumsum(jnp.where(m_eq, 1, 0))
        m_take = m_eq & ((pref + eqc) <= need_eq)
        m = m_gt | m_take
        idxv = lax.broadcasted_iota(jnp.int32, (L,), 0) + i * L
        plsc.store_compressed(ibuf.at[pl.ds(off, L)], idxv, mask=m)
        npop = plsc.all_reduce_population_count(m)
        neq = plsc.all_reduce_population_count(m_take)
        return off + npop[0], eqc + neq[0]
    lax.fori_loop(0, S // L, body, (jnp.int32(0), jnp.int32(0)))


def _topk1_body(scores_hbm, idx_out, map_out, sbuf, kbuf, ibuf, mbuf, hbuf, sem):
    del sem
    wid = lax.axis_index("c") * NS + lax.axis_index("s")

    @pl.when(wid < B)
    def _():
        b = wid
        pltpu.sync_copy(scores_hbm.at[b], sbuf)
        _keys_from_scores(sbuf, kbuf)
        v, need_eq = _threshold(kbuf, hbuf)
        _compact(kbuf, ibuf, v, need_eq)

        def ms(i, _):
            mbuf[pl.ds(i * L, L)] = jnp.full((L,), -1, jnp.int32)
            return 0
        lax.fori_loop(0, S // L, ms, 0)

        def sc(i, _):
            iv = ibuf[pl.ds(i * L, L)]
            pv = lax.broadcasted_iota(jnp.int32, (L,), 0) + i * L
            plsc.store_scatter(mbuf, [iv], pv)
            return 0
        lax.fori_loop(0, K // L, sc, 0)

        pltpu.sync_copy(ibuf.at[pl.ds(0, K)], idx_out.at[b])
        pltpu.sync_copy(mbuf, map_out.at[pl.ds(b * S, S)])


def _topk1(scores):
    return pl.kernel(
        _topk1_body,
        out_type=[
            jax.ShapeDtypeStruct((B, K), jnp.int32),
            jax.ShapeDtypeStruct((B * S,), jnp.int32),
        ],
        mesh=plsc.VectorSubcoreMesh(**_MESH),
        compiler_params=pltpu.CompilerParams(needs_layout_passes=False),
        scratch_types=[
            pltpu.VMEM((S,), jnp.float32),
            pltpu.VMEM((S,), jnp.uint32),
            pltpu.VMEM((K + L,), jnp.int32),
            pltpu.VMEM((S,), jnp.int32),
            pltpu.VMEM((256 * L,), jnp.int32),
            pltpu.SemaphoreType.DMA,
        ],
    )(scores)


def _topk2_body(scores_hbm, idx1_hbm, s2sel_hbm, idx_out,
                sbuf, kbuf, ibuf, vbuf, hbuf, sem):
    del sem
    wid = lax.axis_index("c") * NS + lax.axis_index("s")

    @pl.when(wid < B)
    def _():
        b = wid
        pltpu.sync_copy(scores_hbm.at[b], sbuf)
        pltpu.sync_copy(idx1_hbm.at[b], ibuf.at[pl.ds(0, K)])
        pltpu.sync_copy(s2sel_hbm.at[b], vbuf)

        def upd(i, _):
            iv = ibuf[pl.ds(i * L, L)]
            vv = vbuf[pl.ds(i * L, L)]
            plsc.store_scatter(sbuf, [iv], vv)
            return 0
        lax.fori_loop(0, K // L, upd, 0)

        _keys_from_scores(sbuf, kbuf)
        v, need_eq = _threshold(kbuf, hbuf)
        _compact(kbuf, ibuf, v, need_eq)
        pltpu.sync_copy(ibuf.at[pl.ds(0, K)], idx_out.at[b])


def _topk2(scores2b, idx1, s2sel):
    return pl.kernel(
        _topk2_body,
        out_type=jax.ShapeDtypeStruct((B, K), jnp.int32),
        mesh=plsc.VectorSubcoreMesh(**_MESH),
        compiler_params=pltpu.CompilerParams(needs_layout_passes=False),
        scratch_types=[
            pltpu.VMEM((S,), jnp.float32),
            pltpu.VMEM((S,), jnp.uint32),
            pltpu.VMEM((K + L,), jnp.int32),
            pltpu.VMEM((K,), jnp.float32),
            pltpu.VMEM((256 * L,), jnp.int32),
            pltpu.SemaphoreType.DMA,
        ],
    )(scores2b, idx1, s2sel)


def _gather_body(xf_hbm, idxf_hbm, xg_out, ivb, rows, sem):
    wid = lax.axis_index("c") * NS + lax.axis_index("s")
    base = wid * RPT
    b = wid // (NW // B)
    pltpu.sync_copy(idxf_hbm.at[pl.ds(base, RPT)], ivb)

    def adj(i, _):
        ivb[pl.ds(i * L, L)] = ivb[pl.ds(i * L, L)] + b * S
        return 0
    lax.fori_loop(0, RPT // L, adj, 0)
    pltpu.async_copy(xf_hbm.at[ivb], rows, sem).wait()
    pltpu.sync_copy(rows, xg_out.at[pl.ds(base, RPT)])


def _gather_rows(xf, idxf):
    return pl.kernel(
        _gather_body,
        out_type=jax.ShapeDtypeStruct((B * K, D), jnp.float32),
        mesh=plsc.VectorSubcoreMesh(**_MESH),
        compiler_params=pltpu.CompilerParams(needs_layout_passes=False),
        scratch_types=[
            pltpu.VMEM((RPT,), jnp.int32),
            pltpu.VMEM((RPT, D), jnp.float32),
            pltpu.SemaphoreType.DMA,
        ],
    )(xf, idxf)


def _attn_body(xg_ref, wq_ref, wk_ref, wv_ref, wo_ref, cs_ref, misc_ref,
               h2_ref, s2_ref):
    x = xg_ref[0]
    ssq = jnp.sum(x * x, axis=-1, keepdims=True)
    sel = (x * _prsqrt(ssq * (1.0 / D) + EPS)
           * misc_ref[0][None, :]).astype(jnp.bfloat16)
    q = jnp.dot(sel, wq_ref[...].astype(jnp.bfloat16),
                preferred_element_type=jnp.float32)
    kk = jnp.dot(sel, wk_ref[...].astype(jnp.bfloat16),
                 preferred_element_type=jnp.float32)
    vv = jnp.dot(sel, wv_ref[...].astype(jnp.bfloat16),
                 preferred_element_type=jnp.float32)
    cos = cs_ref[:, :HD // 2]
    sin = cs_ref[:, HD // 2:]

    def rope(m):
        m1 = m[:, :HD // 2]
        m2 = m[:, HD // 2:]
        return jnp.concatenate([m1 * cos - m2 * sin,
                                m2 * cos + m1 * sin], axis=-1)

    kr = [rope(kk[:, g * HD:(g + 1) * HD]) for g in range(KVH)]
    vs = [vv[:, g * HD:(g + 1) * HD] for g in range(KVH)]
    rows = lax.broadcasted_iota(jnp.int32, (K, K), 0)
    cols = lax.broadcasted_iota(jnp.int32, (K, K), 1)
    causal = rows >= cols
    outs = []
    for h in range(H):
        qh = rope(q[:, h * HD:(h + 1) * HD]).astype(jnp.bfloat16)
        g = h // (H // KVH)
        s = lax.dot_general(qh, kr[g].astype(jnp.bfloat16),
                            (((1,), (1,)), ((), ())),
                            preferred_element_type=jnp.float32)
        s = s * (1.0 / np.sqrt(HD))
        s = jnp.where(causal, s, -1e9)
        m = jnp.max(s, axis=-1, keepdims=True)
        e = jnp.exp(s - m)
        p = (e / jnp.sum(e, axis=-1, keepdims=True)).astype(jnp.bfloat16)
        outs.append(jnp.dot(p, vs[g].astype(jnp.bfloat16),
                            preferred_element_type=jnp.float32))
    o = jnp.concatenate(outs, axis=-1).astype(jnp.bfloat16)
    attn = jnp.dot(o, wo_ref[...].astype(jnp.bfloat16),
                   preferred_element_type=jnp.float32)
    h2 = x + attn
    h2_ref[0] = h2
    ssq2 = jnp.sum(h2 * h2, axis=-1, keepdims=True)
    hn = _bf(h2 * _prsqrt(ssq2 * (1.0 / D) + EPS) * misc_ref[1][None, :])
    s2_ref[0, 0] = jnp.sum(hn * _bf(misc_ref[2])[None, :], axis=-1)


def _attention(xg, Wq, Wk, Wv, Wo, cs, misc):
    return pl.pallas_call(
        _attn_body,
        grid=(B,),
        in_specs=[
            pl.BlockSpec((1, K, D), lambda b: (b, 0, 0)),
            pl.BlockSpec((D, H * HD), lambda b: (0, 0)),
            pl.BlockSpec((D, KVH * HD), lambda b: (0, 0)),
            pl.BlockSpec((D, KVH * HD), lambda b: (0, 0)),
            pl.BlockSpec((H * HD, D), lambda b: (0, 0)),
            pl.BlockSpec((K, HD), lambda b: (0, 0)),
            pl.BlockSpec((8, D), lambda b: (0, 0)),
        ],
        out_specs=[
            pl.BlockSpec((1, K, D), lambda b: (b, 0, 0)),
            pl.BlockSpec((1, 1, K), lambda b: (b, 0, 0)),
        ],
        out_shape=[
            jax.ShapeDtypeStruct((B, K, D), jnp.float32),
            jax.ShapeDtypeStruct((B, 1, K), jnp.float32),
        ],
    )(xg, Wq, Wk, Wv, Wo, cs, misc)


def _rows2_body(xf_hbm, h2f_hbm, map_hbm, idx2f_hbm, out_hbm,
                iv2, pv, ivx, ivh, xbuf, hbuf, sem, sem2):
    wid = lax.axis_index("c") * NS + lax.axis_index("s")
    base = wid * RPT
    b = wid // (NW // B)
    CH = 64
    pltpu.sync_copy(idx2f_hbm.at[pl.ds(base, RPT)], iv2)

    def adj(i, _):
        iv2[pl.ds(i * L, L)] = iv2[pl.ds(i * L, L)] + b * S
        return 0
    lax.fori_loop(0, RPT // L, adj, 0)
    pltpu.async_copy(map_hbm.at[iv2], pv, sem).wait()

    for chunk in range(RPT // CH):
        for j in range(CH // L):
            tv = iv2[pl.ds(chunk * CH + j * L, L)]
            mv = pv[pl.ds(chunk * CH + j * L, L)]
            ivx[pl.ds(j * L, L)] = tv
            ivh[pl.ds(j * L, L)] = jnp.maximum(mv, 0) + b * K
        cx = pltpu.async_copy(xf_hbm.at[ivx], xbuf, sem)
        ch = pltpu.async_copy(h2f_hbm.at[ivh], hbuf, sem2)
        cx.wait()
        ch.wait()

        def mix(g, _):
            mvec = pv[pl.ds(chunk * CH + g * L, L)]
            for jj in range(L):
                j = g * L + jj
                msk = jnp.broadcast_to(mvec[jj] >= 0, (L,))

                def dloop(d, _, j=j, msk=msk):
                    for u in range(4):
                        sl = pl.ds((d * 4 + u) * L, L)
                        hv = hbuf[j, sl]
                        xv = xbuf[j, sl]
                        xbuf[j, sl] = jnp.where(msk, hv, 2.0 * xv)
                    return 0
                lax.fori_loop(0, D // (4 * L), dloop, 0)
            return 0
        lax.fori_loop(0, CH // L, mix, 0)
        pltpu.sync_copy(xbuf, out_hbm.at[pl.ds(base + chunk * CH, CH)])


def _gather_rows2(xf, h2f, map1, idx2f):
    CH = 64
    return pl.kernel(
        _rows2_body,
        out_type=jax.ShapeDtypeStruct((B * K, D), jnp.float32),
        mesh=plsc.VectorSubcoreMesh(**_MESH),
        compiler_params=pltpu.CompilerParams(needs_layout_passes=False),
        scratch_types=[
            pltpu.VMEM((RPT,), jnp.int32),
            pltpu.VMEM((RPT,), jnp.int32),
            pltpu.VMEM((CH,), jnp.int32),
            pltpu.VMEM((CH,), jnp.int32),
            pltpu.VMEM((CH, D), jnp.float32),
            pltpu.VMEM((CH, D), jnp.float32),
            pltpu.SemaphoreType.DMA,
            pltpu.SemaphoreType.DMA,
        ],
    )(xf, h2f, map1, idx2f)


def _ffn_body(r_ref, w1_ref, b1_ref, w2_ref, misc_ref, out_ref):
    r = r_ref[0]
    ssq = jnp.sum(r * r, axis=-1, keepdims=True)
    hn = (r * _prsqrt(ssq * (1.0 / D) + EPS)
          * misc_ref[0][None, :]).astype(jnp.bfloat16)
    a = jnp.dot(hn, w1_ref[...].astype(jnp.bfloat16),
                preferred_element_type=jnp.float32)
    a = a + b1_ref[0][None, :]
    sl = (a * (1.0 / (1.0 + jnp.exp(-a)))).astype(jnp.bfloat16)
    f = jnp.dot(sl, w2_ref[...].astype(jnp.bfloat16),
                preferred_element_type=jnp.float32)
    out_ref[0] = r + f + misc_ref[1][None, :]


def _ffn(rows2, W1, b1, W2, misc2):
    KC = 512
    return pl.pallas_call(
        _ffn_body,
        grid=(B, K // KC),
        in_specs=[
            pl.BlockSpec((1, KC, D), lambda b, j: (b, j, 0)),
            pl.BlockSpec((D, DFF), lambda b, j: (0, 0)),
            pl.BlockSpec((1, DFF), lambda b, j: (0, 0)),
            pl.BlockSpec((DFF, D), lambda b, j: (0, 0)),
            pl.BlockSpec((8, D), lambda b, j: (0, 0)),
        ],
        out_specs=pl.BlockSpec((1, KC, D), lambda b, j: (b, j, 0)),
        out_shape=jax.ShapeDtypeStruct((B, K, D), jnp.float32),
    )(rows2, W1, b1, W2, misc2)


_FCH = 32


def _final_body(xf_hbm, h2f_hbm, or2_hbm, idx1f_hbm, idx2f_hbm, out_hbm,
                ib0, ib1, ob0, ob1, ivb, ivc0, ivc1,
                isem0, isem1, osem0, osem1):
    c = lax.axis_index("c")
    s = lax.axis_index("s")
    CH = _FCH
    ROWS = 2 * S // NS
    g0 = (2 * c) * S + s * ROWS
    NQ = ROWS // CH
    ibs, isems = (ib0, ib1), (isem0, isem1)
    obs, osems = (ob0, ob1), (osem0, osem1)

    pltpu.async_copy(xf_hbm.at[pl.ds(g0, CH)], ib0, isem0)

    def pipe(qq, _):
        for j in range(2):
            q = qq * 2 + j
            ib, isem = ibs[j], isems[j]
            ob, osem = obs[j], osems[j]
            nib, nisem = ibs[1 - j], isems[1 - j]

            @pl.when(q + 1 < NQ)
            def _():
                pltpu.async_copy(xf_hbm.at[pl.ds(g0 + (q + 1) * CH, CH)],
                                 nib, nisem)
            pltpu.make_async_copy(xf_hbm.at[pl.ds(g0, CH)], ib, isem).wait()

            @pl.when(q >= 2)
            def _():
                pltpu.make_async_copy(ob, out_hbm.at[pl.ds(g0, CH)],
                                      osem).wait()

            def rowc(r, _, ib=ib, ob=ob):
                for dd in range(D // L):
                    ob[r, pl.ds(dd * L, L)] = ib[r, pl.ds(dd * L, L)] * 4.0
                return 0
            lax.fori_loop(0, CH, rowc, 0)
            pltpu.async_copy(ob, out_hbm.at[pl.ds(g0 + q * CH, CH)], osem)
        return 0
    lax.fori_loop(0, NQ // 2, pipe, 0)
    pltpu.make_async_copy(ob0, out_hbm.at[pl.ds(g0, CH)], osem0).wait()
    pltpu.make_async_copy(ob1, out_hbm.at[pl.ds(g0, CH)], osem1).wait()

    plsc.subcore_barrier()

    NE = 2 * K // NS
    e0 = (2 * c) * K + s * NE
    b = 2 * c + s // (NS // 2)
    NT = NE // CH
    ivcs = (ivc0, ivc1)

    for phase, (src, idx_src, scale) in enumerate(
            ((h2f_hbm, idx1f_hbm, True), (or2_hbm, idx2f_hbm, False))):
        pltpu.sync_copy(idx_src.at[pl.ds(e0, NE)], ivb)

        def adj(i, _):
            ivb[pl.ds(i * L, L)] = ivb[pl.ds(i * L, L)] + b * S
            return 0
        lax.fori_loop(0, NE // L, adj, 0)

        pltpu.async_copy(src.at[pl.ds(e0, CH)], ib0, isem0)
        pltpu.async_copy(src.at[pl.ds(e0 + CH, CH)], ib1, isem1)
        for t in range(NT):
            j = t % 2
            ib, isem, osem, ivc = ibs[j], isems[j], osems[j], ivcs[j]
            pltpu.make_async_copy(src.at[pl.ds(e0, CH)], ib, isem).wait()
            if scale:
                def sc2(r, _, ib=ib):
                    for dd in range(D // L):
                        ib[r, pl.ds(dd * L, L)] = (
                            ib[r, pl.ds(dd * L, L)] * 2.0)
                    return 0
                lax.fori_loop(0, CH, sc2, 0)
            for g in range(CH // L):
                ivc[pl.ds(g * L, L)] = ivb[pl.ds(t * CH + g * L, L)]
            pltpu.async_copy(ib, out_hbm.at[ivc], osem)
            if t + 2 < NT:
                pltpu.make_async_copy(ib, out_hbm.at[ivc], osem).wait()
                pltpu.async_copy(src.at[pl.ds(e0 + (t + 2) * CH, CH)],
                                 ib, isem)
        for j in range(min(2, NT)):
            pltpu.make_async_copy(ibs[j], out_hbm.at[ivcs[j]],
                                  osems[j]).wait()
        if phase == 0:
            plsc.subcore_barrier()


def _final(xf, h2f, or2f, idx1f, idx2f):
    CH = _FCH
    return pl.kernel(
        _final_body,
        out_type=jax.ShapeDtypeStruct((B * S, D), jnp.float32),
        mesh=plsc.VectorSubcoreMesh(**_MESH),
        compiler_params=pltpu.CompilerParams(needs_layout_passes=False),
        scratch_types=[
            pltpu.VMEM((CH, D), jnp.float32),
            pltpu.VMEM((CH, D), jnp.float32),
            pltpu.VMEM((CH, D), jnp.float32),
            pltpu.VMEM((CH, D), jnp.float32),
            pltpu.VMEM((2 * K // NS,), jnp.int32),
            pltpu.VMEM((CH,), jnp.int32),
            pltpu.VMEM((CH,), jnp.int32),
            pltpu.SemaphoreType.DMA,
            pltpu.SemaphoreType.DMA,
            pltpu.SemaphoreType.DMA,
            pltpu.SemaphoreType.DMA,
        ],
    )(xf, h2f, or2f, idx1f, idx2f)


def kernel(hidden_states, seq_norm_w, ffn_norm_w, seq_router_w, ffn_router_w,
           Wq, Wk, Wv, Wo, W1, b1, W2, b2):
    x = hidden_states
    xf = x.reshape(B * S, D)

    vmat = (jnp.zeros((8, D), jnp.float32).at[0].set(seq_norm_w)
            .at[1].set(seq_router_w[:, 0]).at[2].set(ffn_norm_w)
            .at[3].set(ffn_router_w[:, 0]))
    misc = (jnp.zeros((8, D), jnp.float32).at[0].set(seq_norm_w)
            .at[1].set(ffn_norm_w).at[2].set(ffn_router_w[:, 0]))
    misc2 = jnp.zeros((8, D), jnp.float32).at[0].set(ffn_norm_w).at[1].set(b2)

    half = HD // 2
    inv = 1.0 / (THETA ** (jnp.arange(0, half, dtype=jnp.float32) / half))
    ang = jnp.arange(K, dtype=jnp.float32)[:, None] * inv[None, :]
    cs = jnp.concatenate([jnp.cos(ang), jnp.sin(ang)], axis=-1)

    scores1, scores2b = _scores(x, vmat)
    idx1, map1 = _topk1(scores1)
    xg = _gather_rows(xf, idx1.reshape(B * K))
    h2sel, s2sel = _attention(xg.reshape(B, K, D), Wq, Wk, Wv, Wo, cs, misc)
    idx2 = _topk2(scores2b, idx1, s2sel.reshape(B, K))
    h2f = h2sel.reshape(B * K, D)
    rows2 = _gather_rows2(xf, h2f, map1, idx2.reshape(B * K))
    outrows2 = _ffn(rows2.reshape(B, K, D), W1, b1.reshape(1, DFF), W2, misc2)
    out = _final(xf, h2f, outrows2.reshape(B * K, D),
                 idx1.reshape(B * K), idx2.reshape(B * K))
    return out.reshape(B, S, D)


# speedup vs baseline: 3.7962x; 1.0629x over previous
import functools

import jax
import jax.numpy as jnp
import numpy as np
from jax import lax
from jax.experimental import pallas as pl
from jax.experimental.pallas import tpu as pltpu
from jax.experimental.pallas import tpu_sc as plsc

B, S, D = 4, 8192, 768
H, KVH, HD = 12, 4, 64
DFF = 3072
K = 1024
EPS = 1e-6
THETA = 10000.0

NC, NS, L = 2, 16, 16
NW = NC * NS
RPT = (B * K) // NW

_MESH = dict(core_axis_name="c", subcore_axis_name="s", num_cores=NC,
             num_subcores=NS)


def _prsqrt(r):
    y = lax.rsqrt(r)
    y = y * (1.5 - 0.5 * r * y * y)
    return y * (1.5 - 0.5 * r * y * y)

def _bf(z):
    return z.astype(jnp.bfloat16).astype(jnp.float32)


def _scores_body(x_ref, v_ref, s1_ref, s2_ref):
    x = x_ref[...]
    wn1 = v_ref[0]
    r1 = _bf(v_ref[1])
    wn2 = v_ref[2]
    r2 = _bf(v_ref[3])
    ssq = jnp.sum(x * x, axis=-1)
    rs1 = _prsqrt(ssq * (1.0 / D) + EPS)
    h1 = _bf(x * rs1[..., None] * wn1[None, None, :])
    s1_ref[...] = jnp.sum(h1 * r1[None, None, :], axis=-1)
    rs2 = _prsqrt(ssq * (4.0 / D) + EPS)
    h2 = _bf((2.0 * x) * rs2[..., None] * wn2[None, None, :])
    s2_ref[...] = jnp.sum(h2 * r2[None, None, :], axis=-1)


def _scores(x, vmat):
    BS = 512
    return pl.pallas_call(
        _scores_body,
        grid=(S // BS,),
        in_specs=[
            pl.BlockSpec((B, BS, D), lambda i: (0, i, 0)),
            pl.BlockSpec((8, D), lambda i: (0, 0)),
        ],
        out_specs=[
            pl.BlockSpec((B, BS), lambda i: (0, i)),
            pl.BlockSpec((B, BS), lambda i: (0, i)),
        ],
        out_shape=[
            jax.ShapeDtypeStruct((B, S), jnp.float32),
            jax.ShapeDtypeStruct((B, S), jnp.float32),
        ],
    )(x, vmat)


def _keys_from_scores(sbuf, kbuf):
    def cvt(i, _):
        xv = sbuf[pl.ds(i * L, L)]
        u = plsc.bitcast(xv, jnp.uint32)
        neg = (u >> jnp.uint32(31)) == jnp.uint32(1)
        kbuf[pl.ds(i * L, L)] = jnp.where(neg, ~u, u | jnp.uint32(0x80000000))
        return 0
    lax.fori_loop(0, S // L, cvt, 0)


def _threshold(kbuf, hbuf):
    laneoff = lax.broadcasted_iota(jnp.int32, (L,), 0) * 256
    ones = jnp.full((L,), 1, jnp.int32)
    prefix = jnp.uint32(0)
    R = jnp.int32(K)
    for p in range(4):
        shift = jnp.uint32(24 - 8 * p)

        def zero(i, _):
            hbuf[pl.ds(i * L, L)] = jnp.zeros((L,), jnp.int32)
            return 0
        lax.fori_loop(0, (256 * L) // L, zero, 0)

        def build(i, _, shift=shift, prefix=prefix, p=p):
            kv = kbuf[pl.ds(i * L, L)]
            byte = ((kv >> shift) & jnp.uint32(0xFF)).astype(jnp.int32)
            idx = laneoff + byte
            if p == 0:
                plsc.addupdate_scatter(hbuf, [idx], ones)
            else:
                act = (kv >> (shift + jnp.uint32(8))) == prefix
                plsc.addupdate_scatter(hbuf, [idx], ones, mask=act)
            return 0
        lax.fori_loop(0, S // L, build, 0)

        bstar = jnp.int32(0)
        cnt_above = jnp.int32(0)
        cum_higher = jnp.int32(0)
        for g in range(15, -1, -1):
            bsum = hbuf[pl.ds(g * L, L)]
            for lane in range(1, L):
                bsum = bsum + hbuf[pl.ds(lane * 256 + g * L, L)]
            sfx = lax.rev(plsc.cumsum(lax.rev(bsum, (0,))), (0,))
            cgt = cum_higher + sfx - bsum
            cond = (cgt < R) & ((cgt + bsum) >= R)
            lanes = lax.broadcasted_iota(jnp.int32, (L,), 0) + g * L
            bstar = bstar + jnp.sum(jnp.where(cond, lanes, 0))
            cnt_above = cnt_above + jnp.sum(jnp.where(cond, cgt, 0))
            cum_higher = cum_higher + jnp.sum(bsum)
        R = R - cnt_above
        prefix = (prefix << jnp.uint32(8)) | bstar.astype(jnp.uint32)
    return prefix, R


def _compact(kbuf, ibuf, v, need_eq):
    def body(i, carry):
        off, eqc = carry
        kv = kbuf[pl.ds(i * L, L)]
        m_gt = kv > v
        m_eq = kv == v
        pref = plsc.cumsum(jnp.where(m_eq, 1, 0))
        m_take = m_eq & ((pref + eqc) <= need_eq)
        m = m_gt | m_take
        idxv = lax.broadcasted_iota(jnp.int32, (L,), 0) + i * L
        plsc.store_compressed(ibuf.at[pl.ds(off, L)], idxv, mask=m)
        npop = plsc.all_reduce_population_count(m)
        neq = plsc.all_reduce_population_count(m_take)
        return off + npop[0], eqc + neq[0]
    lax.fori_loop(0, S // L, body, (jnp.int32(0), jnp.int32(0)))


def _topk1_body(scores_hbm, idx_out, map_out, sbuf, kbuf, ibuf, mbuf, hbuf, sem):
    del sem
    wid = lax.axis_index("c") * NS + lax.axis_index("s")

    @pl.when(wid < B)
    def _():
        b = wid
        pltpu.sync_copy(scores_hbm.at[b], sbuf)
        _keys_from_scores(sbuf, kbuf)
        v, need_eq = _threshold(kbuf, hbuf)
        _compact(kbuf, ibuf, v, need_eq)

        def ms(i, _):
            mbuf[pl.ds(i * L, L)] = jnp.full((L,), -1, jnp.int32)
            return 0
        lax.fori_loop(0, S // L, ms, 0)

        def sc(i, _):
            iv = ibuf[pl.ds(i * L, L)]
            pv = lax.broadcasted_iota(jnp.int32, (L,), 0) + i * L
            plsc.store_scatter(mbuf, [iv], pv)
            return 0
        lax.fori_loop(0, K // L, sc, 0)

        pltpu.sync_copy(ibuf.at[pl.ds(0, K)], idx_out.at[b])
        pltpu.sync_copy(mbuf, map_out.at[pl.ds(b * S, S)])


def _topk1(scores):
    return pl.kernel(
        _topk1_body,
        out_type=[
            jax.ShapeDtypeStruct((B, K), jnp.int32),
            jax.ShapeDtypeStruct((B * S,), jnp.int32),
        ],
        mesh=plsc.VectorSubcoreMesh(**_MESH),
        compiler_params=pltpu.CompilerParams(needs_layout_passes=False),
        scratch_types=[
            pltpu.VMEM((S,), jnp.float32),
            pltpu.VMEM((S,), jnp.uint32),
            pltpu.VMEM((K + L,), jnp.int32),
            pltpu.VMEM((S,), jnp.int32),
            pltpu.VMEM((256 * L,), jnp.int32),
            pltpu.SemaphoreType.DMA,
        ],
    )(scores)


def _topk2_body(scores_hbm, idx1_hbm, s2sel_hbm, idx_out,
                sbuf, kbuf, ibuf, vbuf, hbuf, sem):
    del sem
    wid = lax.axis_index("c") * NS + lax.axis_index("s")

    @pl.when(wid < B)
    def _():
        b = wid
        pltpu.sync_copy(scores_hbm.at[b], sbuf)
        pltpu.sync_copy(idx1_hbm.at[b], ibuf.at[pl.ds(0, K)])
        pltpu.sync_copy(s2sel_hbm.at[b], vbuf)

        def upd(i, _):
            iv = ibuf[pl.ds(i * L, L)]
            vv = vbuf[pl.ds(i * L, L)]
            plsc.store_scatter(sbuf, [iv], vv)
            return 0
        lax.fori_loop(0, K // L, upd, 0)

        _keys_from_scores(sbuf, kbuf)
        v, need_eq = _threshold(kbuf, hbuf)
        _compact(kbuf, ibuf, v, need_eq)
        pltpu.sync_copy(ibuf.at[pl.ds(0, K)], idx_out.at[b])


def _topk2(scores2b, idx1, s2sel):
    return pl.kernel(
        _topk2_body,
        out_type=jax.ShapeDtypeStruct((B, K), jnp.int32),
        mesh=plsc.VectorSubcoreMesh(**_MESH),
        compiler_params=pltpu.CompilerParams(needs_layout_passes=False),
        scratch_types=[
            pltpu.VMEM((S,), jnp.float32),
            pltpu.VMEM((S,), jnp.uint32),
            pltpu.VMEM((K + L,), jnp.int32),
            pltpu.VMEM((K,), jnp.float32),
            pltpu.VMEM((256 * L,), jnp.int32),
            pltpu.SemaphoreType.DMA,
        ],
    )(scores2b, idx1, s2sel)


def _gather_body(xf_hbm, idxf_hbm, xg_out, ivb, rows, sem):
    wid = lax.axis_index("c") * NS + lax.axis_index("s")
    base = wid * RPT
    b = wid // (NW // B)
    pltpu.sync_copy(idxf_hbm.at[pl.ds(base, RPT)], ivb)

    def adj(i, _):
        ivb[pl.ds(i * L, L)] = ivb[pl.ds(i * L, L)] + b * S
        return 0
    lax.fori_loop(0, RPT // L, adj, 0)
    pltpu.async_copy(xf_hbm.at[ivb], rows, sem).wait()
    pltpu.sync_copy(rows, xg_out.at[pl.ds(base, RPT)])


def _gather_rows(xf, idxf):
    return pl.kernel(
        _gather_body,
        out_type=jax.ShapeDtypeStruct((B * K, D), jnp.float32),
        mesh=plsc.VectorSubcoreMesh(**_MESH),
        compiler_params=pltpu.CompilerParams(needs_layout_passes=False),
        scratch_types=[
            pltpu.VMEM((RPT,), jnp.int32),
            pltpu.VMEM((RPT, D), jnp.float32),
            pltpu.SemaphoreType.DMA,
        ],
    )(xf, idxf)


def _attn_body(xg_ref, wq_ref, wk_ref, wv_ref, wo_ref, cs_ref, misc_ref,
               h2_ref, s2_ref):
    x = xg_ref[0]
    ssq = jnp.sum(x * x, axis=-1, keepdims=True)
    sel = (x * _prsqrt(ssq * (1.0 / D) + EPS)
           * misc_ref[0][None, :]).astype(jnp.bfloat16)
    q = jnp.dot(sel, wq_ref[...].astype(jnp.bfloat16),
                preferred_element_type=jnp.float32)
    kk = jnp.dot(sel, wk_ref[...].astype(jnp.bfloat16),
                 preferred_element_type=jnp.float32)
    vv = jnp.dot(sel, wv_ref[...].astype(jnp.bfloat16),
                 preferred_element_type=jnp.float32)
    cos = cs_ref[:, :HD // 2]
    sin = cs_ref[:, HD // 2:]

    def rope(m):
        m1 = m[:, :HD // 2]
        m2 = m[:, HD // 2:]
        return jnp.concatenate([m1 * cos - m2 * sin,
                                m2 * cos + m1 * sin], axis=-1)

    kr = [rope(kk[:, g * HD:(g + 1) * HD]).astype(jnp.bfloat16)
          for g in range(KVH)]
    vs = [vv[:, g * HD:(g + 1) * HD].astype(jnp.bfloat16)
          for g in range(KVH)]
    QB = 256
    rows = lax.broadcasted_iota(jnp.int32, (QB, QB), 0)
    cols = lax.broadcasted_iota(jnp.int32, (QB, QB), 1)
    diag_mask = rows >= cols
    outs = []
    for h in range(H):
        qh = rope(q[:, h * HD:(h + 1) * HD]).astype(jnp.bfloat16)
        g = h // (H // KVH)
        oblocks = []
        for qi in range(K // QB):
            ncols = (qi + 1) * QB
            s = lax.dot_general(qh[qi * QB:(qi + 1) * QB], kr[g][:ncols],
                                (((1,), (1,)), ((), ())),
                                preferred_element_type=jnp.float32)
            s = s * (1.0 / np.sqrt(HD))
            s = jnp.concatenate(
                [s[:, :qi * QB],
                 jnp.where(diag_mask, s[:, qi * QB:], -1e9)], axis=-1) \
                if qi else jnp.where(diag_mask, s, -1e9)
            m = jnp.max(s, axis=-1, keepdims=True)
            e = jnp.exp(s - m)
            p = (e / jnp.sum(e, axis=-1, keepdims=True)).astype(jnp.bfloat16)
            oblocks.append(jnp.dot(p, vs[g][:ncols],
                                   preferred_element_type=jnp.float32))
        outs.append(jnp.concatenate(oblocks, axis=0))
    o = jnp.concatenate(outs, axis=-1).astype(jnp.bfloat16)
    attn = jnp.dot(o, wo_ref[...].astype(jnp.bfloat16),
                   preferred_element_type=jnp.float32)
    h2 = x + attn
    h2_ref[0] = h2
    ssq2 = jnp.sum(h2 * h2, axis=-1, keepdims=True)
    hn = _bf(h2 * _prsqrt(ssq2 * (1.0 / D) + EPS) * misc_ref[1][None, :])
    s2_ref[0, 0] = jnp.sum(hn * _bf(misc_ref[2])[None, :], axis=-1)


def _attention(xg, Wq, Wk, Wv, Wo, cs, misc):
    return pl.pallas_call(
        _attn_body,
        grid=(B,),
        in_specs=[
            pl.BlockSpec((1, K, D), lambda b: (b, 0, 0)),
            pl.BlockSpec((D, H * HD), lambda b: (0, 0)),
            pl.BlockSpec((D, KVH * HD), lambda b: (0, 0)),
            pl.BlockSpec((D, KVH * HD), lambda b: (0, 0)),
            pl.BlockSpec((H * HD, D), lambda b: (0, 0)),
            pl.BlockSpec((K, HD), lambda b: (0, 0)),
            pl.BlockSpec((8, D), lambda b: (0, 0)),
        ],
        out_specs=[
            pl.BlockSpec((1, K, D), lambda b: (b, 0, 0)),
            pl.BlockSpec((1, 1, K), lambda b: (b, 0, 0)),
        ],
        out_shape=[
            jax.ShapeDtypeStruct((B, K, D), jnp.float32),
            jax.ShapeDtypeStruct((B, 1, K), jnp.float32),
        ],
    )(xg, Wq, Wk, Wv, Wo, cs, misc)


def _rows2_body(xf_hbm, h2f_hbm, map_hbm, idx2f_hbm, out_hbm,
                iv2, pv, ivx, ivh, xbuf, hbuf, sem, sem2):
    wid = lax.axis_index("c") * NS + lax.axis_index("s")
    base = wid * RPT
    b = wid // (NW // B)
    CH = 64
    pltpu.sync_copy(idx2f_hbm.at[pl.ds(base, RPT)], iv2)

    def adj(i, _):
        iv2[pl.ds(i * L, L)] = iv2[pl.ds(i * L, L)] + b * S
        return 0
    lax.fori_loop(0, RPT // L, adj, 0)
    pltpu.async_copy(map_hbm.at[iv2], pv, sem).wait()

    for chunk in range(RPT // CH):
        for j in range(CH // L):
            tv = iv2[pl.ds(chunk * CH + j * L, L)]
            mv = pv[pl.ds(chunk * CH + j * L, L)]
            ivx[pl.ds(j * L, L)] = tv
            ivh[pl.ds(j * L, L)] = jnp.maximum(mv, 0) + b * K
        cx = pltpu.async_copy(xf_hbm.at[ivx], xbuf, sem)
        ch = pltpu.async_copy(h2f_hbm.at[ivh], hbuf, sem2)
        cx.wait()
        ch.wait()

        def mix(g, _):
            mvec = pv[pl.ds(chunk * CH + g * L, L)]
            for jj in range(L):
                j = g * L + jj
                msk = jnp.broadcast_to(mvec[jj] >= 0, (L,))

                def dloop(d, _, j=j, msk=msk):
                    for u in range(4):
                        sl = pl.ds((d * 4 + u) * L, L)
                        hv = hbuf[j, sl]
                        xv = xbuf[j, sl]
                        xbuf[j, sl] = jnp.where(msk, hv, 2.0 * xv)
                    return 0
                lax.fori_loop(0, D // (4 * L), dloop, 0)
            return 0
        lax.fori_loop(0, CH // L, mix, 0)
        pltpu.sync_copy(xbuf, out_hbm.at[pl.ds(base + chunk * CH, CH)])


def _gather_rows2(xf, h2f, map1, idx2f):
    CH = 64
    return pl.kernel(
        _rows2_body,
        out_type=jax.ShapeDtypeStruct((B * K, D), jnp.float32),
        mesh=plsc.VectorSubcoreMesh(**_MESH),
        compiler_params=pltpu.CompilerParams(needs_layout_passes=False),
        scratch_types=[
            pltpu.VMEM((RPT,), jnp.int32),
            pltpu.VMEM((RPT,), jnp.int32),
            pltpu.VMEM((CH,), jnp.int32),
            pltpu.VMEM((CH,), jnp.int32),
            pltpu.VMEM((CH, D), jnp.float32),
            pltpu.VMEM((CH, D), jnp.float32),
            pltpu.SemaphoreType.DMA,
            pltpu.SemaphoreType.DMA,
        ],
    )(xf, h2f, map1, idx2f)


def _ffn_body(r_ref, w1_ref, b1_ref, w2_ref, misc_ref, out_ref):
    r = r_ref[0]
    ssq = jnp.sum(r * r, axis=-1, keepdims=True)
    hn = (r * _prsqrt(ssq * (1.0 / D) + EPS)
          * misc_ref[0][None, :]).astype(jnp.bfloat16)
    a = jnp.dot(hn, w1_ref[...].astype(jnp.bfloat16),
                preferred_element_type=jnp.float32)
    a = a + b1_ref[0][None, :]
    sl = (a * (1.0 / (1.0 + jnp.exp(-a)))).astype(jnp.bfloat16)
    f = jnp.dot(sl, w2_ref[...].astype(jnp.bfloat16),
                preferred_element_type=jnp.float32)
    out_ref[0] = r + f + misc_ref[1][None, :]


def _ffn(rows2, W1, b1, W2, misc2):
    KC = 512
    return pl.pallas_call(
        _ffn_body,
        grid=(B, K // KC),
        in_specs=[
            pl.BlockSpec((1, KC, D), lambda b, j: (b, j, 0)),
            pl.BlockSpec((D, DFF), lambda b, j: (0, 0)),
            pl.BlockSpec((1, DFF), lambda b, j: (0, 0)),
            pl.BlockSpec((DFF, D), lambda b, j: (0, 0)),
            pl.BlockSpec((8, D), lambda b, j: (0, 0)),
        ],
        out_specs=pl.BlockSpec((1, KC, D), lambda b, j: (b, j, 0)),
        out_shape=jax.ShapeDtypeStruct((B, K, D), jnp.float32),
    )(rows2, W1, b1, W2, misc2)


_FCH = 32


def _final_body(xf_hbm, h2f_hbm, or2_hbm, idx1f_hbm, idx2f_hbm, out_hbm,
                ib0, ib1, ob0, ob1, ivb, ivc0, ivc1,
                isem0, isem1, osem0, osem1):
    c = lax.axis_index("c")
    s = lax.axis_index("s")
    CH = _FCH
    ROWS = 2 * S // NS
    g0 = (2 * c) * S + s * ROWS
    NQ = ROWS // CH
    ibs, isems = (ib0, ib1), (isem0, isem1)
    obs, osems = (ob0, ob1), (osem0, osem1)

    pltpu.async_copy(xf_hbm.at[pl.ds(g0, CH)], ib0, isem0)

    def pipe(qq, _):
        for j in range(2):
            q = qq * 2 + j
            ib, isem = ibs[j], isems[j]
            ob, osem = obs[j], osems[j]
            nib, nisem = ibs[1 - j], isems[1 - j]

            @pl.when(q + 1 < NQ)
            def _():
                pltpu.async_copy(xf_hbm.at[pl.ds(g0 + (q + 1) * CH, CH)],
                                 nib, nisem)
            pltpu.make_async_copy(xf_hbm.at[pl.ds(g0, CH)], ib, isem).wait()

            @pl.when(q >= 2)
            def _():
                pltpu.make_async_copy(ob, out_hbm.at[pl.ds(g0, CH)],
                                      osem).wait()

            def rowc(r, _, ib=ib, ob=ob):
                for dd in range(D // L):
                    ob[r, pl.ds(dd * L, L)] = ib[r, pl.ds(dd * L, L)] * 4.0
                return 0
            lax.fori_loop(0, CH, rowc, 0)
            pltpu.async_copy(ob, out_hbm.at[pl.ds(g0 + q * CH, CH)], osem)
        return 0
    lax.fori_loop(0, NQ // 2, pipe, 0)
    pltpu.make_async_copy(ob0, out_hbm.at[pl.ds(g0, CH)], osem0).wait()
    pltpu.make_async_copy(ob1, out_hbm.at[pl.ds(g0, CH)], osem1).wait()

    plsc.subcore_barrier()

    NE = 2 * K // NS
    e0 = (2 * c) * K + s * NE
    b = 2 * c + s // (NS // 2)
    NT = NE // CH
    ivcs = (ivc0, ivc1)

    for phase, (src, idx_src, scale) in enumerate(
            ((h2f_hbm, idx1f_hbm, True), (or2_hbm, idx2f_hbm, False))):
        pltpu.sync_copy(idx_src.at[pl.ds(e0, NE)], ivb)

        def adj(i, _):
            ivb[pl.ds(i * L, L)] = ivb[pl.ds(i * L, L)] + b * S
            return 0
        lax.fori_loop(0, NE // L, adj, 0)

        pltpu.async_copy(src.at[pl.ds(e0, CH)], ib0, isem0)
        pltpu.async_copy(src.at[pl.ds(e0 + CH, CH)], ib1, isem1)
        for t in range(NT):
            j = t % 2
            ib, isem, osem, ivc = ibs[j], isems[j], osems[j], ivcs[j]
            pltpu.make_async_copy(src.at[pl.ds(e0, CH)], ib, isem).wait()
            if scale:
                def sc2(r, _, ib=ib):
                    for dd in range(D // L):
                        ib[r, pl.ds(dd * L, L)] = (
                            ib[r, pl.ds(dd * L, L)] * 2.0)
                    return 0
                lax.fori_loop(0, CH, sc2, 0)
            for g in range(CH // L):
                ivc[pl.ds(g * L, L)] = ivb[pl.ds(t * CH + g * L, L)]
            pltpu.async_copy(ib, out_hbm.at[ivc], osem)
            if t + 2 < NT:
                pltpu.make_async_copy(ib, out_hbm.at[ivc], osem).wait()
                pltpu.async_copy(src.at[pl.ds(e0 + (t + 2) * CH, CH)],
                                 ib, isem)
        for j in range(min(2, NT)):
            pltpu.make_async_copy(ibs[j], out_hbm.at[ivcs[j]],
                                  osems[j]).wait()
        if phase == 0:
            plsc.subcore_barrier()


def _final(xf, h2f, or2f, idx1f, idx2f):
    CH = _FCH
    return pl.kernel(
        _final_body,
        out_type=jax.ShapeDtypeStruct((B * S, D), jnp.float32),
        mesh=plsc.VectorSubcoreMesh(**_MESH),
        compiler_params=pltpu.CompilerParams(needs_layout_passes=False),
        scratch_types=[
            pltpu.VMEM((CH, D), jnp.float32),
            pltpu.VMEM((CH, D), jnp.float32),
            pltpu.VMEM((CH, D), jnp.float32),
            pltpu.VMEM((CH, D), jnp.float32),
            pltpu.VMEM((2 * K // NS,), jnp.int32),
            pltpu.VMEM((CH,), jnp.int32),
            pltpu.VMEM((CH,), jnp.int32),
            pltpu.SemaphoreType.DMA,
            pltpu.SemaphoreType.DMA,
            pltpu.SemaphoreType.DMA,
            pltpu.SemaphoreType.DMA,
        ],
    )(xf, h2f, or2f, idx1f, idx2f)


def kernel(hidden_states, seq_norm_w, ffn_norm_w, seq_router_w, ffn_router_w,
           Wq, Wk, Wv, Wo, W1, b1, W2, b2):
    x = hidden_states
    xf = x.reshape(B * S, D)

    vmat = (jnp.zeros((8, D), jnp.float32).at[0].set(seq_norm_w)
            .at[1].set(seq_router_w[:, 0]).at[2].set(ffn_norm_w)
            .at[3].set(ffn_router_w[:, 0]))
    misc = (jnp.zeros((8, D), jnp.float32).at[0].set(seq_norm_w)
            .at[1].set(ffn_norm_w).at[2].set(ffn_router_w[:, 0]))
    misc2 = jnp.zeros((8, D), jnp.float32).at[0].set(ffn_norm_w).at[1].set(b2)

    half = HD // 2
    inv = 1.0 / (THETA ** (jnp.arange(0, half, dtype=jnp.float32) / half))
    ang = jnp.arange(K, dtype=jnp.float32)[:, None] * inv[None, :]
    cs = jnp.concatenate([jnp.cos(ang), jnp.sin(ang)], axis=-1)

    scores1, scores2b = _scores(x, vmat)
    idx1, map1 = _topk1(scores1)
    xg = _gather_rows(xf, idx1.reshape(B * K))
    h2sel, s2sel = _attention(xg.reshape(B, K, D), Wq, Wk, Wv, Wo, cs, misc)
    idx2 = _topk2(scores2b, idx1, s2sel.reshape(B, K))
    h2f = h2sel.reshape(B * K, D)
    rows2 = _gather_rows2(xf, h2f, map1, idx2.reshape(B * K))
    outrows2 = _ffn(rows2.reshape(B, K, D), W1, b1.reshape(1, DFF), W2, misc2)
    out = _final(xf, h2f, outrows2.reshape(B * K, D),
                 idx1.reshape(B * K), idx2.reshape(B * K))
    return out.reshape(B, S, D)


# speedup vs baseline: 3.8634x; 1.0177x over previous
import functools

import jax
import jax.numpy as jnp
import numpy as np
from jax import lax
from jax.experimental import pallas as pl
from jax.experimental.pallas import tpu as pltpu
from jax.experimental.pallas import tpu_sc as plsc

B, S, D = 4, 8192, 768
H, KVH, HD = 12, 4, 64
DFF = 3072
K = 1024
EPS = 1e-6
THETA = 10000.0

NC, NS, L = 2, 16, 16
NW = NC * NS
RPT = (B * K) // NW

_MESH = dict(core_axis_name="c", subcore_axis_name="s", num_cores=NC,
             num_subcores=NS)


def _prsqrt(r):
    y = lax.rsqrt(r)
    y = y * (1.5 - 0.5 * r * y * y)
    return y * (1.5 - 0.5 * r * y * y)

def _bf(z):
    return z.astype(jnp.bfloat16).astype(jnp.float32)


def _scores_body(x_ref, v_ref, s1_ref, s2_ref):
    x = x_ref[...]
    wn1 = v_ref[0]
    r1 = _bf(v_ref[1])
    wn2 = v_ref[2]
    r2 = _bf(v_ref[3])
    ssq = jnp.sum(x * x, axis=-1)
    rs1 = _prsqrt(ssq * (1.0 / D) + EPS)
    h1 = _bf(x * rs1[..., None] * wn1[None, None, :])
    s1_ref[...] = jnp.sum(h1 * r1[None, None, :], axis=-1)
    rs2 = _prsqrt(ssq * (4.0 / D) + EPS)
    h2 = _bf((2.0 * x) * rs2[..., None] * wn2[None, None, :])
    s2_ref[...] = jnp.sum(h2 * r2[None, None, :], axis=-1)


def _scores(x, vmat):
    BS = 512
    return pl.pallas_call(
        _scores_body,
        grid=(S // BS,),
        in_specs=[
            pl.BlockSpec((B, BS, D), lambda i: (0, i, 0)),
            pl.BlockSpec((8, D), lambda i: (0, 0)),
        ],
        out_specs=[
            pl.BlockSpec((B, BS), lambda i: (0, i)),
            pl.BlockSpec((B, BS), lambda i: (0, i)),
        ],
        out_shape=[
            jax.ShapeDtypeStruct((B, S), jnp.float32),
            jax.ShapeDtypeStruct((B, S), jnp.float32),
        ],
    )(x, vmat)


def _keys_from_scores(sbuf, kbuf):
    def cvt(i, _):
        for j in range(4):
            sl = pl.ds((i * 4 + j) * L, L)
            u = plsc.bitcast(sbuf[sl], jnp.uint32)
            neg = (u >> jnp.uint32(31)) == jnp.uint32(1)
            kbuf[sl] = jnp.where(neg, ~u, u | jnp.uint32(0x80000000))
        return 0
    lax.fori_loop(0, S // (4 * L), cvt, 0)


def _threshold(kbuf, hbuf):
    laneoff = lax.broadcasted_iota(jnp.int32, (L,), 0) * 256
    ones = jnp.full((L,), 1, jnp.int32)
    prefix = jnp.uint32(0)
    R = jnp.int32(K)
    for p in range(4):
        shift = jnp.uint32(24 - 8 * p)

        def zero(i, _):
            for j in range(4):
                hbuf[pl.ds((i * 4 + j) * L, L)] = jnp.zeros((L,), jnp.int32)
            return 0
        lax.fori_loop(0, (256 * L) // (4 * L), zero, 0)

        def build(i, _, shift=shift, prefix=prefix, p=p):
            for j in range(4):
                kv = kbuf[pl.ds((i * 4 + j) * L, L)]
                byte = ((kv >> shift) & jnp.uint32(0xFF)).astype(jnp.int32)
                idx = laneoff + byte
                if p == 0:
                    plsc.addupdate_scatter(hbuf, [idx], ones)
                else:
                    act = (kv >> (shift + jnp.uint32(8))) == prefix
                    plsc.addupdate_scatter(hbuf, [idx], ones, mask=act)
            return 0
        lax.fori_loop(0, S // (4 * L), build, 0)

        bstar = jnp.int32(0)
        cnt_above = jnp.int32(0)
        cum_higher = jnp.int32(0)
        for g in range(15, -1, -1):
            bsum = hbuf[pl.ds(g * L, L)]
            for lane in range(1, L):
                bsum = bsum + hbuf[pl.ds(lane * 256 + g * L, L)]
            sfx = lax.rev(plsc.cumsum(lax.rev(bsum, (0,))), (0,))
            cgt = cum_higher + sfx - bsum
            cond = (cgt < R) & ((cgt + bsum) >= R)
            lanes = lax.broadcasted_iota(jnp.int32, (L,), 0) + g * L
            bstar = bstar + jnp.sum(jnp.where(cond, lanes, 0))
            cnt_above = cnt_above + jnp.sum(jnp.where(cond, cgt, 0))
            cum_higher = cum_higher + jnp.sum(bsum)
        R = R - cnt_above
        prefix = (prefix << jnp.uint32(8)) | bstar.astype(jnp.uint32)
    return prefix, R


def _compact(kbuf, ibuf, v, need_eq):
    def body(i, carry):
        off, eqc = carry
        for j in range(2):
            kv = kbuf[pl.ds((i * 2 + j) * L, L)]
            m_gt = kv > v
            m_eq = kv == v
            pref = plsc.cumsum(jnp.where(m_eq, 1, 0))
            m_take = m_eq & ((pref + eqc) <= need_eq)
            m = m_gt | m_take
            idxv = lax.broadcasted_iota(jnp.int32, (L,), 0) + (i * 2 + j) * L
            plsc.store_compressed(ibuf.at[pl.ds(off, L)], idxv, mask=m)
            npop = plsc.all_reduce_population_count(m)
            neq = plsc.all_reduce_population_count(m_take)
            off = off + npop[0]
            eqc = eqc + neq[0]
        return off, eqc
    lax.fori_loop(0, S // (2 * L), body, (jnp.int32(0), jnp.int32(0)))


def _topk1_body(scores_hbm, idx_out, map_out, sbuf, kbuf, ibuf, mbuf, hbuf, sem):
    del sem
    wid = lax.axis_index("c") * NS + lax.axis_index("s")

    @pl.when(wid < B)
    def _():
        b = wid
        pltpu.sync_copy(scores_hbm.at[b], sbuf)
        _keys_from_scores(sbuf, kbuf)
        v, need_eq = _threshold(kbuf, hbuf)
        _compact(kbuf, ibuf, v, need_eq)

        def ms(i, _):
            for j in range(4):
                mbuf[pl.ds((i * 4 + j) * L, L)] = jnp.full((L,), -1,
                                                           jnp.int32)
            return 0
        lax.fori_loop(0, S // (4 * L), ms, 0)

        def sc(i, _):
            iv = ibuf[pl.ds(i * L, L)]
            pv = lax.broadcasted_iota(jnp.int32, (L,), 0) + i * L
            plsc.store_scatter(mbuf, [iv], pv)
            return 0
        lax.fori_loop(0, K // L, sc, 0)

        pltpu.sync_copy(ibuf.at[pl.ds(0, K)], idx_out.at[b])
        pltpu.sync_copy(mbuf, map_out.at[pl.ds(b * S, S)])


def _topk1(scores):
    return pl.kernel(
        _topk1_body,
        out_type=[
            jax.ShapeDtypeStruct((B, K), jnp.int32),
            jax.ShapeDtypeStruct((B * S,), jnp.int32),
        ],
        mesh=plsc.VectorSubcoreMesh(**_MESH),
        compiler_params=pltpu.CompilerParams(needs_layout_passes=False),
        scratch_types=[
            pltpu.VMEM((S,), jnp.float32),
            pltpu.VMEM((S,), jnp.uint32),
            pltpu.VMEM((K + L,), jnp.int32),
            pltpu.VMEM((S,), jnp.int32),
            pltpu.VMEM((256 * L,), jnp.int32),
            pltpu.SemaphoreType.DMA,
        ],
    )(scores)


def _topk2_body(scores_hbm, idx1_hbm, s2sel_hbm, idx_out,
                sbuf, kbuf, ibuf, vbuf, hbuf, sem):
    del sem
    wid = lax.axis_index("c") * NS + lax.axis_index("s")

    @pl.when(wid < B)
    def _():
        b = wid
        pltpu.sync_copy(scores_hbm.at[b], sbuf)
        pltpu.sync_copy(idx1_hbm.at[b], ibuf.at[pl.ds(0, K)])
        pltpu.sync_copy(s2sel_hbm.at[b], vbuf)

        def upd(i, _):
            iv = ibuf[pl.ds(i * L, L)]
            vv = vbuf[pl.ds(i * L, L)]
            plsc.store_scatter(sbuf, [iv], vv)
            return 0
        lax.fori_loop(0, K // L, upd, 0)

        _keys_from_scores(sbuf, kbuf)
        v, need_eq = _threshold(kbuf, hbuf)
        _compact(kbuf, ibuf, v, need_eq)
        pltpu.sync_copy(ibuf.at[pl.ds(0, K)], idx_out.at[b])


def _topk2(scores2b, idx1, s2sel):
    return pl.kernel(
        _topk2_body,
        out_type=jax.ShapeDtypeStruct((B, K), jnp.int32),
        mesh=plsc.VectorSubcoreMesh(**_MESH),
        compiler_params=pltpu.CompilerParams(needs_layout_passes=False),
        scratch_types=[
            pltpu.VMEM((S,), jnp.float32),
            pltpu.VMEM((S,), jnp.uint32),
            pltpu.VMEM((K + L,), jnp.int32),
            pltpu.VMEM((K,), jnp.float32),
            pltpu.VMEM((256 * L,), jnp.int32),
            pltpu.SemaphoreType.DMA,
        ],
    )(scores2b, idx1, s2sel)


def _gather_body(xf_hbm, idxf_hbm, xg_out, ivb, rows, sem):
    wid = lax.axis_index("c") * NS + lax.axis_index("s")
    base = wid * RPT
    b = wid // (NW // B)
    pltpu.sync_copy(idxf_hbm.at[pl.ds(base, RPT)], ivb)

    def adj(i, _):
        ivb[pl.ds(i * L, L)] = ivb[pl.ds(i * L, L)] + b * S
        return 0
    lax.fori_loop(0, RPT // L, adj, 0)
    pltpu.async_copy(xf_hbm.at[ivb], rows, sem).wait()
    pltpu.sync_copy(rows, xg_out.at[pl.ds(base, RPT)])


def _gather_rows(xf, idxf):
    return pl.kernel(
        _gather_body,
        out_type=jax.ShapeDtypeStruct((B * K, D), jnp.float32),
        mesh=plsc.VectorSubcoreMesh(**_MESH),
        compiler_params=pltpu.CompilerParams(needs_layout_passes=False),
        scratch_types=[
            pltpu.VMEM((RPT,), jnp.int32),
            pltpu.VMEM((RPT, D), jnp.float32),
            pltpu.SemaphoreType.DMA,
        ],
    )(xf, idxf)


def _attn_body(xg_ref, wq_ref, wk_ref, wv_ref, wo_ref, cs_ref, misc_ref,
               h2_ref, s2_ref):
    x = xg_ref[0]
    ssq = jnp.sum(x * x, axis=-1, keepdims=True)
    sel = (x * _prsqrt(ssq * (1.0 / D) + EPS)
           * misc_ref[0][None, :]).astype(jnp.bfloat16)
    q = jnp.dot(sel, wq_ref[...].astype(jnp.bfloat16),
                preferred_element_type=jnp.float32)
    kk = jnp.dot(sel, wk_ref[...].astype(jnp.bfloat16),
                 preferred_element_type=jnp.float32)
    vv = jnp.dot(sel, wv_ref[...].astype(jnp.bfloat16),
                 preferred_element_type=jnp.float32)
    cos = cs_ref[:, :HD // 2]
    sin = cs_ref[:, HD // 2:]

    def rope(m):
        m1 = m[:, :HD // 2]
        m2 = m[:, HD // 2:]
        return jnp.concatenate([m1 * cos - m2 * sin,
                                m2 * cos + m1 * sin], axis=-1)

    kr = [rope(kk[:, g * HD:(g + 1) * HD]).astype(jnp.bfloat16)
          for g in range(KVH)]
    vs = [vv[:, g * HD:(g + 1) * HD].astype(jnp.bfloat16)
          for g in range(KVH)]
    QB = 256
    rows = lax.broadcasted_iota(jnp.int32, (QB, QB), 0)
    cols = lax.broadcasted_iota(jnp.int32, (QB, QB), 1)
    diag_mask = rows >= cols
    outs = []
    for h in range(H):
        qh = rope(q[:, h * HD:(h + 1) * HD]).astype(jnp.bfloat16)
        g = h // (H // KVH)
        oblocks = []
        for qi in range(K // QB):
            ncols = (qi + 1) * QB
            s = lax.dot_general(qh[qi * QB:(qi + 1) * QB], kr[g][:ncols],
                                (((1,), (1,)), ((), ())),
                                preferred_element_type=jnp.float32)
            s = s * (1.0 / np.sqrt(HD))
            s = jnp.concatenate(
                [s[:, :qi * QB],
                 jnp.where(diag_mask, s[:, qi * QB:], -1e9)], axis=-1) \
                if qi else jnp.where(diag_mask, s, -1e9)
            m = jnp.max(s, axis=-1, keepdims=True)
            e = jnp.exp(s - m)
            p = (e / jnp.sum(e, axis=-1, keepdims=True)).astype(jnp.bfloat16)
            oblocks.append(jnp.dot(p, vs[g][:ncols],
                                   preferred_element_type=jnp.float32))
        outs.append(jnp.concatenate(oblocks, axis=0))
    o = jnp.concatenate(outs, axis=-1).astype(jnp.bfloat16)
    attn = jnp.dot(o, wo_ref[...].astype(jnp.bfloat16),
                   preferred_element_type=jnp.float32)
    h2 = x + attn
    h2_ref[0] = h2
    ssq2 = jnp.sum(h2 * h2, axis=-1, keepdims=True)
    hn = _bf(h2 * _prsqrt(ssq2 * (1.0 / D) + EPS) * misc_ref[1][None, :])
    s2_ref[0, 0] = jnp.sum(hn * _bf(misc_ref[2])[None, :], axis=-1)


def _attention(xg, Wq, Wk, Wv, Wo, cs, misc):
    return pl.pallas_call(
        _attn_body,
        grid=(B,),
        in_specs=[
            pl.BlockSpec((1, K, D), lambda b: (b, 0, 0)),
            pl.BlockSpec((D, H * HD), lambda b: (0, 0)),
            pl.BlockSpec((D, KVH * HD), lambda b: (0, 0)),
            pl.BlockSpec((D, KVH * HD), lambda b: (0, 0)),
            pl.BlockSpec((H * HD, D), lambda b: (0, 0)),
            pl.BlockSpec((K, HD), lambda b: (0, 0)),
            pl.BlockSpec((8, D), lambda b: (0, 0)),
        ],
        out_specs=[
            pl.BlockSpec((1, K, D), lambda b: (b, 0, 0)),
            pl.BlockSpec((1, 1, K), lambda b: (b, 0, 0)),
        ],
        out_shape=[
            jax.ShapeDtypeStruct((B, K, D), jnp.float32),
            jax.ShapeDtypeStruct((B, 1, K), jnp.float32),
        ],
    )(xg, Wq, Wk, Wv, Wo, cs, misc)


def _rows2_body(xf_hbm, h2f_hbm, map_hbm, idx2f_hbm, out_hbm,
                iv2, pv, ivx, ivh, xbuf, hbuf, sem, sem2):
    wid = lax.axis_index("c") * NS + lax.axis_index("s")
    base = wid * RPT
    b = wid // (NW // B)
    CH = 64
    pltpu.sync_copy(idx2f_hbm.at[pl.ds(base, RPT)], iv2)

    def adj(i, _):
        iv2[pl.ds(i * L, L)] = iv2[pl.ds(i * L, L)] + b * S
        return 0
    lax.fori_loop(0, RPT // L, adj, 0)
    pltpu.async_copy(map_hbm.at[iv2], pv, sem).wait()

    for chunk in range(RPT // CH):
        for j in range(CH // L):
            tv = iv2[pl.ds(chunk * CH + j * L, L)]
            mv = pv[pl.ds(chunk * CH + j * L, L)]
            ivx[pl.ds(j * L, L)] = tv
            ivh[pl.ds(j * L, L)] = jnp.maximum(mv, 0) + b * K
        cx = pltpu.async_copy(xf_hbm.at[ivx], xbuf, sem)
        ch = pltpu.async_copy(h2f_hbm.at[ivh], hbuf, sem2)
        cx.wait()
        ch.wait()

        def mix(g, _):
            mvec = pv[pl.ds(chunk * CH + g * L, L)]
            for jj in range(L):
                j = g * L + jj
                msk = jnp.broadcast_to(mvec[jj] >= 0, (L,))

                def dloop(d, _, j=j, msk=msk):
                    for u in range(4):
                        sl = pl.ds((d * 4 + u) * L, L)
                        hv = hbuf[j, sl]
                        xv = xbuf[j, sl]
                        xbuf[j, sl] = jnp.where(msk, hv, 2.0 * xv)
                    return 0
                lax.fori_loop(0, D // (4 * L), dloop, 0)
            return 0
        lax.fori_loop(0, CH // L, mix, 0)
        pltpu.sync_copy(xbuf, out_hbm.at[pl.ds(base + chunk * CH, CH)])


def _gather_rows2(xf, h2f, map1, idx2f):
    CH = 64
    return pl.kernel(
        _rows2_body,
        out_type=jax.ShapeDtypeStruct((B * K, D), jnp.float32),
        mesh=plsc.VectorSubcoreMesh(**_MESH),
        compiler_params=pltpu.CompilerParams(needs_layout_passes=False),
        scratch_types=[
            pltpu.VMEM((RPT,), jnp.int32),
            pltpu.VMEM((RPT,), jnp.int32),
            pltpu.VMEM((CH,), jnp.int32),
            pltpu.VMEM((CH,), jnp.int32),
            pltpu.VMEM((CH, D), jnp.float32),
            pltpu.VMEM((CH, D), jnp.float32),
            pltpu.SemaphoreType.DMA,
            pltpu.SemaphoreType.DMA,
        ],
    )(xf, h2f, map1, idx2f)


def _ffn_body(r_ref, w1_ref, b1_ref, w2_ref, misc_ref, out_ref):
    r = r_ref[0]
    ssq = jnp.sum(r * r, axis=-1, keepdims=True)
    hn = (r * _prsqrt(ssq * (1.0 / D) + EPS)
          * misc_ref[0][None, :]).astype(jnp.bfloat16)
    a = jnp.dot(hn, w1_ref[...].astype(jnp.bfloat16),
                preferred_element_type=jnp.float32)
    a = a + b1_ref[0][None, :]
    sl = (a * (1.0 / (1.0 + jnp.exp(-a)))).astype(jnp.bfloat16)
    f = jnp.dot(sl, w2_ref[...].astype(jnp.bfloat16),
                preferred_element_type=jnp.float32)
    out_ref[0] = r + f + misc_ref[1][None, :]


def _ffn(rows2, W1, b1, W2, misc2):
    KC = 512
    return pl.pallas_call(
        _ffn_body,
        grid=(B, K // KC),
        in_specs=[
            pl.BlockSpec((1, KC, D), lambda b, j: (b, j, 0)),
            pl.BlockSpec((D, DFF), lambda b, j: (0, 0)),
            pl.BlockSpec((1, DFF), lambda b, j: (0, 0)),
            pl.BlockSpec((DFF, D), lambda b, j: (0, 0)),
            pl.BlockSpec((8, D), lambda b, j: (0, 0)),
        ],
        out_specs=pl.BlockSpec((1, KC, D), lambda b, j: (b, j, 0)),
        out_shape=jax.ShapeDtypeStruct((B, K, D), jnp.float32),
    )(rows2, W1, b1, W2, misc2)


_FCH = 32


def _final_body(xf_hbm, h2f_hbm, or2_hbm, idx1f_hbm, idx2f_hbm, out_hbm,
                ib0, ib1, ob0, ob1, ivb, ivc0, ivc1,
                isem0, isem1, osem0, osem1):
    c = lax.axis_index("c")
    s = lax.axis_index("s")
    CH = _FCH
    ROWS = 2 * S // NS
    g0 = (2 * c) * S + s * ROWS
    NQ = ROWS // CH
    ibs, isems = (ib0, ib1), (isem0, isem1)
    obs, osems = (ob0, ob1), (osem0, osem1)

    pltpu.async_copy(xf_hbm.at[pl.ds(g0, CH)], ib0, isem0)

    def pipe(qq, _):
        for j in range(2):
            q = qq * 2 + j
            ib, isem = ibs[j], isems[j]
            ob, osem = obs[j], osems[j]
            nib, nisem = ibs[1 - j], isems[1 - j]

            @pl.when(q + 1 < NQ)
            def _():
                pltpu.async_copy(xf_hbm.at[pl.ds(g0 + (q + 1) * CH, CH)],
                                 nib, nisem)
            pltpu.make_async_copy(xf_hbm.at[pl.ds(g0, CH)], ib, isem).wait()

            @pl.when(q >= 2)
            def _():
                pltpu.make_async_copy(ob, out_hbm.at[pl.ds(g0, CH)],
                                      osem).wait()

            def rowc(r, _, ib=ib, ob=ob):
                for dd in range(D // L):
                    ob[r, pl.ds(dd * L, L)] = ib[r, pl.ds(dd * L, L)] * 4.0
                return 0
            lax.fori_loop(0, CH, rowc, 0)
            pltpu.async_copy(ob, out_hbm.at[pl.ds(g0 + q * CH, CH)], osem)
        return 0
    lax.fori_loop(0, NQ // 2, pipe, 0)
    pltpu.make_async_copy(ob0, out_hbm.at[pl.ds(g0, CH)], osem0).wait()
    pltpu.make_async_copy(ob1, out_hbm.at[pl.ds(g0, CH)], osem1).wait()

    plsc.subcore_barrier()

    NE = 2 * K // NS
    e0 = (2 * c) * K + s * NE
    b = 2 * c + s // (NS // 2)
    NT = NE // CH
    ivcs = (ivc0, ivc1)

    for phase, (src, idx_src, scale) in enumerate(
            ((h2f_hbm, idx1f_hbm, True), (or2_hbm, idx2f_hbm, False))):
        pltpu.sync_copy(idx_src.at[pl.ds(e0, NE)], ivb)

        def adj(i, _):
            ivb[pl.ds(i * L, L)] = ivb[pl.ds(i * L, L)] + b * S
            return 0
        lax.fori_loop(0, NE // L, adj, 0)

        pltpu.async_copy(src.at[pl.ds(e0, CH)], ib0, isem0)
        pltpu.async_copy(src.at[pl.ds(e0 + CH, CH)], ib1, isem1)
        for t in range(NT):
            j = t % 2
            ib, isem, osem, ivc = ibs[j], isems[j], osems[j], ivcs[j]
            pltpu.make_async_copy(src.at[pl.ds(e0, CH)], ib, isem).wait()
            if scale:
                def sc2(r, _, ib=ib):
                    for dd in range(D // L):
                        ib[r, pl.ds(dd * L, L)] = (
                            ib[r, pl.ds(dd * L, L)] * 2.0)
                    return 0
                lax.fori_loop(0, CH, sc2, 0)
            for g in range(CH // L):
                ivc[pl.ds(g * L, L)] = ivb[pl.ds(t * CH + g * L, L)]
            pltpu.async_copy(ib, out_hbm.at[ivc], osem)
            if t + 2 < NT:
                pltpu.make_async_copy(ib, out_hbm.at[ivc], osem).wait()
                pltpu.async_copy(src.at[pl.ds(e0 + (t + 2) * CH, CH)],
                                 ib, isem)
        for j in range(min(2, NT)):
            pltpu.make_async_copy(ibs[j], out_hbm.at[ivcs[j]],
                                  osems[j]).wait()
        if phase == 0:
            plsc.subcore_barrier()


def _final(xf, h2f, or2f, idx1f, idx2f):
    CH = _FCH
    return pl.kernel(
        _final_body,
        out_type=jax.ShapeDtypeStruct((B * S, D), jnp.float32),
        mesh=plsc.VectorSubcoreMesh(**_MESH),
        compiler_params=pltpu.CompilerParams(needs_layout_passes=False),
        scratch_types=[
            pltpu.VMEM((CH, D), jnp.float32),
            pltpu.VMEM((CH, D), jnp.float32),
            pltpu.VMEM((CH, D), jnp.float32),
            pltpu.VMEM((CH, D), jnp.float32),
            pltpu.VMEM((2 * K // NS,), jnp.int32),
            pltpu.VMEM((CH,), jnp.int32),
            pltpu.VMEM((CH,), jnp.int32),
            pltpu.SemaphoreType.DMA,
            pltpu.SemaphoreType.DMA,
            pltpu.SemaphoreType.DMA,
            pltpu.SemaphoreType.DMA,
        ],
    )(xf, h2f, or2f, idx1f, idx2f)


def kernel(hidden_states, seq_norm_w, ffn_norm_w, seq_router_w, ffn_router_w,
           Wq, Wk, Wv, Wo, W1, b1, W2, b2):
    x = hidden_states
    xf = x.reshape(B * S, D)

    vmat = (jnp.zeros((8, D), jnp.float32).at[0].set(seq_norm_w)
            .at[1].set(seq_router_w[:, 0]).at[2].set(ffn_norm_w)
            .at[3].set(ffn_router_w[:, 0]))
    misc = (jnp.zeros((8, D), jnp.float32).at[0].set(seq_norm_w)
            .at[1].set(ffn_norm_w).at[2].set(ffn_router_w[:, 0]))
    misc2 = jnp.zeros((8, D), jnp.float32).at[0].set(ffn_norm_w).at[1].set(b2)

    half = HD // 2
    inv = 1.0 / (THETA ** (jnp.arange(0, half, dtype=jnp.float32) / half))
    ang = jnp.arange(K, dtype=jnp.float32)[:, None] * inv[None, :]
    cs = jnp.concatenate([jnp.cos(ang), jnp.sin(ang)], axis=-1)

    scores1, scores2b = _scores(x, vmat)
    idx1, map1 = _topk1(scores1)
    xg = _gather_rows(xf, idx1.reshape(B * K))
    h2sel, s2sel = _attention(xg.reshape(B, K, D), Wq, Wk, Wv, Wo, cs, misc)
    idx2 = _topk2(scores2b, idx1, s2sel.reshape(B, K))
    h2f = h2sel.reshape(B * K, D)
    rows2 = _gather_rows2(xf, h2f, map1, idx2.reshape(B * K))
    outrows2 = _ffn(rows2.reshape(B, K, D), W1, b1.reshape(1, DFF), W2, misc2)
    out = _final(xf, h2f, outrows2.reshape(B * K, D),
                 idx1.reshape(B * K), idx2.reshape(B * K))
    return out.reshape(B, S, D)
